# baseline (speedup 1.0000x reference)
import functools

import jax
import jax.numpy as jnp
import numpy as np
from jax import lax
from jax.experimental import pallas as pl
from jax.experimental.pallas import tpu as pltpu
from jax.experimental.pallas import tpu_sc as plsc

N_NODES = 10000
N_EDGES = 160000
N_ANGLES = 400000
F = 128
L_MAX = 4
N_MAX = 4
CUTOFF = 5.0
CUT3 = 4.0
NUM_EL = 108
NBLOCKS = 4

R = 256
EP = 160512
AP = 400384
NP = 10240
GE = EP // R
GA = AP // R
GN = NP // R

_PREC = jax.lax.Precision.HIGHEST


def _swish(x):
    return x / (1.0 + jnp.exp(-x))


def _sigmoid(x):
    return 1.0 / (1.0 + jnp.exp(-x))


def _poly_cutoff(r, c):
    t = jnp.clip(r / c, 0.0, 1.0)
    return 1.0 - 6.0 * t ** 5 + 15.0 * t ** 4 - 10.0 * t ** 3


def _bessel_cols(r, cutoff):
    r_ = r + 1e-8
    s = np.sqrt(2.0 / cutoff).astype(np.float32)
    return [s * jnp.sin((n + 1) * np.float32(np.pi) * r_ / cutoff) / r_
            for n in range(N_MAX + 1)]


def _legendre_cols(c):
    polys = [jnp.ones_like(c), c]
    for l in range(2, L_MAX + 1):
        polys.append(((2 * l - 1) * c * polys[-1] - (l - 1) * polys[-2]) / l)
    return polys


def _k_emb(atomic_col, emb_pad):
    def body(a_ref, w_ref, o_ref):
        a = a_ref[...]
        lanes = lax.broadcasted_iota(jnp.int32, (1, F), 1)
        oh = (a == lanes).astype(jnp.float32)
        o_ref[...] = jnp.dot(oh, w_ref[...], precision=_PREC)

    return pl.pallas_call(
        body,
        grid=(GN,),
        in_specs=[pl.BlockSpec((R, 1), lambda i: (i, 0)),
                  pl.BlockSpec((F, F), lambda i: (0, 0))],
        out_specs=pl.BlockSpec((R, F), lambda i: (i, 0)),
        out_shape=jax.ShapeDtypeStruct((NP, F), jnp.float32),
    )(atomic_col, emb_pad)


def _k_enc(dist_col, enc_W_pad, enc_b, We3_0):
    def body(r_ref, w_ref, b_ref, w3_ref, e_ref, t_ref):
        r = r_ref[...]
        e0 = _bessel_cols(r, CUTOFF)
        w = w_ref[...]
        acc = b_ref[...]
        for k in range(N_MAX + 1):
            acc = acc + e0[k] * w[k:k + 1, :]
        e = _swish(acc)
        e_ref[...] = e
        t_ref[...] = _swish(jnp.dot(e, w3_ref[...], precision=_PREC))

    return pl.pallas_call(
        body,
        grid=(GE,),
        in_specs=[pl.BlockSpec((R, 1), lambda i: (i, 0)),
                  pl.BlockSpec((8, F), lambda i: (0, 0)),
                  pl.BlockSpec((1, F), lambda i: (0, 0)),
                  pl.BlockSpec((F, F), lambda i: (0, 0))],
        out_specs=[pl.BlockSpec((R, F), lambda i: (i, 0)),
                   pl.BlockSpec((R, F), lambda i: (i, 0))],
        out_shape=[jax.ShapeDtypeStruct((EP, F), jnp.float32),
                   jax.ShapeDtypeStruct((EP, F), jnp.float32)],
    )(dist_col, enc_W_pad, enc_b, We3_0)


def _k_ang(norm_col, cos_col):
    def body(r_ref, c_ref, o_ref):
        r = r_ref[...]
        c = c_ref[...]
        rad = _bessel_cols(r, CUT3)
        leg = _legendre_cols(c)
        fc3 = _poly_cutoff(r, CUT3)
        lanes = lax.broadcasted_iota(jnp.int32, (1, 32), 1)
        acc = jnp.zeros((R, 32), jnp.float32)
        for l in range(L_MAX + 1):
            for n in range(N_MAX + 1):
                k = l * (N_MAX + 1) + n
                mask = (lanes == k).astype(jnp.float32)
                acc = acc + (leg[l] * rad[n] * fc3) * mask
        o_ref[...] = acc

    return pl.pallas_call(
        body,
        grid=(GA,),
        in_specs=[pl.BlockSpec((R, 1), lambda i: (i, 0)),
                  pl.BlockSpec((R, 1), lambda i: (i, 0))],
        out_specs=pl.BlockSpec((R, 32), lambda i: (i, 0)),
        out_shape=jax.ShapeDtypeStruct((AP, 32), jnp.float32),
    )(norm_col, cos_col)


def _k_msg3_cumsum(ang, g, Wang_pad, Ltri):
    def body(a_ref, g_ref, w_ref, l_ref, c_ref, carry):
        i = pl.program_id(0)

        @pl.when(i == 0)
        def _():
            carry[...] = jnp.zeros((8, F), jnp.float32)

        a = jnp.dot(a_ref[...], w_ref[...], precision=_PREC)
        msg = a * g_ref[...]
        cv = carry[0:1, :]
        c_ref[...] = cv + jnp.dot(l_ref[...], msg, precision=_PREC)
        carry[0:1, :] = cv + jnp.sum(msg, axis=0, keepdims=True)

    return pl.pallas_call(
        body,
        grid=(GA,),
        in_specs=[pl.BlockSpec((R, 32), lambda i: (i, 0)),
                  pl.BlockSpec((R, F), lambda i: (i, 0)),
                  pl.BlockSpec((32, F), lambda i: (0, 0)),
                  pl.BlockSpec((R, R), lambda i: (0, 0))],
        out_specs=pl.BlockSpec((R, F), lambda i: (i, 0)),
        out_shape=jax.ShapeDtypeStruct((AP, F), jnp.float32),
        scratch_shapes=[pltpu.VMEM((8, F), jnp.float32)],
    )(ang, g, Wang_pad, Ltri)


def _k_edge_node(Ga, Gb, e, xs, xd, dist_col, W3o, Wedge, Wnode, Weg_pad,
                 Wng_pad, Ltri, We3n, emit_t):
    def body(ga_ref, gb_ref, e_ref, xs_ref, xd_ref, r_ref, w3o_ref, we_ref,
             wn_ref, weg_ref, wng_ref, l_ref, we3_ref, *out_and_scratch):
        if emit_t:
            e2_ref, c_ref, t_ref, carry = out_and_scratch
        else:
            e2_ref, c_ref, carry = out_and_scratch
        i = pl.program_id(0)

        @pl.when(i == 0)
        def _():
            carry[...] = jnp.zeros((8, F), jnp.float32)

        agg3 = gb_ref[...] - ga_ref[...]
        e1 = e_ref[...] + _swish(jnp.dot(agg3, w3o_ref[...], precision=_PREC))

        r = r_ref[...]
        e0 = _bessel_cols(r, CUTOFF)
        fc = _poly_cutoff(r, CUTOFF)
        weg = weg_ref[...]
        wng = wng_ref[...]
        gate_e = e0[0] * weg[0:1, :]
        gate_n = e0[0] * wng[0:1, :]
        for k in range(1, N_MAX + 1):
            gate_e = gate_e + e0[k] * weg[k:k + 1, :]
            gate_n = gate_n + e0[k] * wng[k:k + 1, :]

        xs = xs_ref[...]
        xd = xd_ref[...]
        we = we_ref[...]
        wn = wn_ref[...]
        arg_e = (jnp.dot(xs, we[0:F, :], precision=_PREC)
                 + jnp.dot(xd, we[F:2 * F, :], precision=_PREC)
                 + jnp.dot(e1, we[2 * F:3 * F, :], precision=_PREC))
        e2 = e1 + _swish(arg_e) * gate_e * fc
        arg_n = (jnp.dot(xs, wn[0:F, :], precision=_PREC)
                 + jnp.dot(xd, wn[F:2 * F, :], precision=_PREC)
                 + jnp.dot(e1, wn[2 * F:3 * F, :], precision=_PREC))
        msg = _swish(arg_n) * gate_n * fc

        cv = carry[0:1, :]
        c_ref[...] = cv + jnp.dot(l_ref[...], msg, precision=_PREC)
        carry[0:1, :] = cv + jnp.sum(msg, axis=0, keepdims=True)
        e2_ref[...] = e2
        if emit_t:
            t_ref[...] = _swish(jnp.dot(e2, we3_ref[...], precision=_PREC))

    n_out = 3 if emit_t else 2
    return pl.pallas_call(
        body,
        grid=(GE,),
        in_specs=[pl.BlockSpec((R, F), lambda i: (i, 0)),
                  pl.BlockSpec((R, F), lambda i: (i, 0)),
                  pl.BlockSpec((R, F), lambda i: (i, 0)),
                  pl.BlockSpec((R, F), lambda i: (i, 0)),
                  pl.BlockSpec((R, F), lambda i: (i, 0)),
                  pl.BlockSpec((R, 1), lambda i: (i, 0)),
                  pl.BlockSpec((F, F), lambda i: (0, 0)),
                  pl.BlockSpec((3 * F, F), lambda i: (0, 0)),
                  pl.BlockSpec((3 * F, F), lambda i: (0, 0)),
                  pl.BlockSpec((8, F), lambda i: (0, 0)),
                  pl.BlockSpec((8, F), lambda i: (0, 0)),
                  pl.BlockSpec((R, R), lambda i: (0, 0)),
                  pl.BlockSpec((F, F), lambda i: (0, 0))],
        out_specs=[pl.BlockSpec((R, F), lambda i: (i, 0))] * n_out,
        out_shape=[jax.ShapeDtypeStruct((EP, F), jnp.float32)] * n_out,
        scratch_shapes=[pltpu.VMEM((8, F), jnp.float32)],
    )(Ga, Gb, e, xs, xd, dist_col, W3o, Wedge, Wnode, Weg_pad, Wng_pad,
      Ltri, We3n)


def _k_xupd(x, Pa, Pb):
    def body(x_ref, a_ref, b_ref, o_ref):
        o_ref[...] = x_ref[...] + b_ref[...] - a_ref[...]

    return pl.pallas_call(
        body,
        grid=(GN,),
        in_specs=[pl.BlockSpec((R, F), lambda i: (i, 0))] * 3,
        out_specs=pl.BlockSpec((R, F), lambda i: (i, 0)),
        out_shape=jax.ShapeDtypeStruct((NP, F), jnp.float32),
    )(x, Pa, Pb)


def _k_out(x, eW1, eb1, eW2, eb2, eW3_row):
    def body(x_ref, w1_ref, b1_ref, w2_ref, b2_ref, w3_ref, o_ref):
        h = _swish(jnp.dot(x_ref[...], w1_ref[...], precision=_PREC)
                   + b1_ref[...])
        h = _swish(jnp.dot(h, w2_ref[...], precision=_PREC) + b2_ref[...])
        o_ref[...] = jnp.sum(h * w3_ref[...], axis=1, keepdims=True)

    return pl.pallas_call(
        body,
        grid=(GN,),
        in_specs=[pl.BlockSpec((R, F), lambda i: (i, 0)),
                  pl.BlockSpec((F, F), lambda i: (0, 0)),
                  pl.BlockSpec((1, F), lambda i: (0, 0)),
                  pl.BlockSpec((F, F), lambda i: (0, 0)),
                  pl.BlockSpec((1, F), lambda i: (0, 0)),
                  pl.BlockSpec((1, F), lambda i: (0, 0))],
        out_specs=pl.BlockSpec((R, 1), lambda i: (i, 0)),
        out_shape=jax.ShapeDtypeStruct((NP, 1), jnp.float32),
    )(x, eW1, eb1, eW2, eb2, eW3_row)


_NW = 32
_CH = 128


def _gather_one_call(table, idx):
    B = idx.shape[0]
    per = B // _NW
    halfp = per // 2
    nfull = halfp // _CH
    rem = halfp - nfull * _CH
    mesh = plsc.VectorSubcoreMesh(core_axis_name="c", subcore_axis_name="s")

    @functools.partial(
        pl.kernel, mesh=mesh,
        out_type=jax.ShapeDtypeStruct((B, F), jnp.float32),
        scratch_types=[pltpu.VMEM((_CH,), jnp.int32),
                       pltpu.VMEM((_CH, F), jnp.float32),
                       pltpu.VMEM((_CH,), jnp.int32),
                       pltpu.VMEM((_CH, F), jnp.float32),
                       pltpu.SemaphoreType.DMA,
                       pltpu.SemaphoreType.DMA],
    )
    def k(tab, ih, oh, iva, rva, ivb, rvb, sa, sb):
        wid = lax.axis_index("s") * 2 + lax.axis_index("c")
        base = wid * per

        def do(offa, offb, n):
            pltpu.sync_copy(ih.at[pl.ds(offa, n)], iva.at[pl.ds(0, n)])
            cpa = pltpu.async_copy(tab.at[iva.at[pl.ds(0, n)]],
                                   rva.at[pl.ds(0, n)], sa)
            pltpu.sync_copy(ih.at[pl.ds(offb, n)], ivb.at[pl.ds(0, n)])
            cpb = pltpu.async_copy(tab.at[ivb.at[pl.ds(0, n)]],
                                   rvb.at[pl.ds(0, n)], sb)
            cpa.wait()
            pltpu.sync_copy(rva.at[pl.ds(0, n)], oh.at[pl.ds(offa, n)])
            cpb.wait()
            pltpu.sync_copy(rvb.at[pl.ds(0, n)], oh.at[pl.ds(offb, n)])

        def body(i, _):
            do(base + i * _CH, base + halfp + i * _CH, _CH)
            return ()

        lax.fori_loop(0, nfull, body, ())
        if rem:
            do(base + nfull * _CH, base + halfp + nfull * _CH, rem)

    return k(table, idx)


def _gather_pair_call(table, idx_a, idx_b):
    B = idx_a.shape[0]
    per = B // _NW
    nfull = per // _CH
    rem = per - nfull * _CH
    mesh = plsc.VectorSubcoreMesh(core_axis_name="c", subcore_axis_name="s")

    @functools.partial(
        pl.kernel, mesh=mesh,
        out_type=(jax.ShapeDtypeStruct((B, F), jnp.float32),
                  jax.ShapeDtypeStruct((B, F), jnp.float32)),
        scratch_types=[pltpu.VMEM((_CH,), jnp.int32),
                       pltpu.VMEM((_CH, F), jnp.float32),
                       pltpu.VMEM((_CH,), jnp.int32),
                       pltpu.VMEM((_CH, F), jnp.float32),
                       pltpu.SemaphoreType.DMA,
                       pltpu.SemaphoreType.DMA],
    )
    def k(tab, ia, ib, oa, ob, iva, rva, ivb, rvb, sa, sb):
        wid = lax.axis_index("s") * 2 + lax.axis_index("c")
        base = wid * per

        def do(off, n):
            pltpu.sync_copy(ia.at[pl.ds(off, n)], iva.at[pl.ds(0, n)])
            cpa = pltpu.async_copy(tab.at[iva.at[pl.ds(0, n)]],
                                   rva.at[pl.ds(0, n)], sa)
            pltpu.sync_copy(ib.at[pl.ds(off, n)], ivb.at[pl.ds(0, n)])
            cpb = pltpu.async_copy(tab.at[ivb.at[pl.ds(0, n)]],
                                   rvb.at[pl.ds(0, n)], sb)
            cpa.wait()
            pltpu.sync_copy(rva.at[pl.ds(0, n)], oa.at[pl.ds(off, n)])
            cpb.wait()
            pltpu.sync_copy(rvb.at[pl.ds(0, n)], ob.at[pl.ds(off, n)])

        def body(i, _):
            do(base + i * _CH, _CH)
            return ()

        lax.fori_loop(0, nfull, body, ())
        if rem:
            do(base + nfull * _CH, rem)

    return k(table, idx_a, idx_b)


def _pad1(a, n, val):
    return jnp.concatenate(
        [a, jnp.full((n - a.shape[0],), val, a.dtype)])


def kernel(atomic_numbers, edge_index, edge_dist, three_body_indices, norm_ik,
           three_body_cos_angles, total_num_bonds, total_num_angles, params):
    p = params
    f32 = jnp.float32
    tbi0 = three_body_indices[:, 0].astype(jnp.int32)
    tbi1 = three_body_indices[:, 1].astype(jnp.int32)
    src = edge_index[0].astype(jnp.int32)
    dst = edge_index[1].astype(jnp.int32)

    eperm = jnp.argsort(dst)
    dst_s = dst[eperm]
    src_s = src[eperm]
    dist_s = edge_dist[eperm]
    inv_eperm = jnp.zeros((N_EDGES,), jnp.int32).at[eperm].set(
        jnp.arange(N_EDGES, dtype=jnp.int32))
    tbi0r = inv_eperm[tbi0]
    tbi1r = inv_eperm[tbi1]
    aperm = jnp.argsort(tbi0r)
    tbi1_s = tbi1r[aperm]
    norm_s = norm_ik[aperm]
    cos_s = three_body_cos_angles[aperm]

    cnt_a = jnp.zeros((N_EDGES,), jnp.int32).at[tbi0r].add(1)
    csa = jnp.cumsum(cnt_a)
    rsA_a = _pad1(jnp.concatenate([jnp.zeros((1,), jnp.int32), csa[:-1]]),
                  EP, N_ANGLES)
    rsB_a = _pad1(csa, EP, N_ANGLES)
    cnt_n = jnp.zeros((N_NODES,), jnp.int32).at[dst].add(1)
    csn = jnp.cumsum(cnt_n)
    rsA_n = _pad1(jnp.concatenate([jnp.zeros((1,), jnp.int32), csn[:-1]]),
                  NP, N_EDGES)
    rsB_n = _pad1(csn, NP, N_EDGES)

    dist_col = _pad1(dist_s.astype(f32), EP, 10.0)[:, None]
    norm_col = _pad1(norm_s.astype(f32), AP, 10.0)[:, None]
    cos_col = _pad1(cos_s.astype(f32), AP, 0.0)[:, None]
    atomic_col = _pad1(atomic_numbers.astype(jnp.int32), NP, 0)[:, None]
    tbi1_p = _pad1(tbi1_s, AP, 0)
    src_p = _pad1(src_s, EP, 0)
    dst_p = _pad1(dst_s, EP, 0)

    emb_pad = jnp.zeros((F, F), f32).at[:NUM_EL].set(p["emb"].astype(f32))
    enc_W_pad = jnp.zeros((8, F), f32).at[:N_MAX + 1].set(
        p["enc_W"].astype(f32))
    enc_b = p["enc_b"].astype(f32)[None, :]
    Ltri = jnp.asarray(np.tril(np.ones((R, R), np.float32), -1))

    blocks = p["blocks"]
    Wang_pads = [jnp.zeros((32, F), f32).at[:25].set(b["Wang"].astype(f32))
                 for b in blocks]
    Weg_pads = [jnp.zeros((8, F), f32).at[:N_MAX + 1].set(
        b["Weg"].astype(f32)) for b in blocks]
    Wng_pads = [jnp.zeros((8, F), f32).at[:N_MAX + 1].set(
        b["Wng"].astype(f32)) for b in blocks]

    x = _k_emb(atomic_col, emb_pad)
    e, t = _k_enc(dist_col, enc_W_pad, enc_b, blocks[0]["We3"].astype(f32))
    ang = _k_ang(norm_col, cos_col)

    for b in range(NBLOCKS):
        blk = blocks[b]
        g = _gather_one_call(t, tbi1_p)
        C = _k_msg3_cumsum(ang, g, Wang_pads[b], Ltri)
        Ga, Gb = _gather_pair_call(C, rsA_a, rsB_a)
        xs, xd = _gather_pair_call(x, src_p, dst_p)
        emit_t = b < NBLOCKS - 1
        We3n = (blocks[b + 1]["We3"] if emit_t else blocks[0]["We3"]).astype(f32)
        outs = _k_edge_node(Ga, Gb, e, xs, xd, dist_col, blk["W3o"].astype(f32),
                            blk["Wedge"].astype(f32), blk["Wnode"].astype(f32),
                            Weg_pads[b], Wng_pads[b], Ltri, We3n, emit_t)
        if emit_t:
            e, Cmsg, t = outs
        else:
            e, Cmsg = outs
        Pa, Pb = _gather_pair_call(Cmsg, rsA_n, rsB_n)
        x = _k_xupd(x, Pa, Pb)

    energy = _k_out(x, p["eW1"].astype(f32), p["eb1"].astype(f32)[None, :],
                    p["eW2"].astype(f32), p["eb2"].astype(f32)[None, :],
                    p["eW3"].astype(f32)[:, 0][None, :])
    return energy[:N_NODES] + p["eb3"].astype(f32)[None, :]


# speedup vs baseline: 1.2461x; 1.2461x over previous
import functools

import jax
import jax.numpy as jnp
import numpy as np
from jax import lax
from jax.experimental import pallas as pl
from jax.experimental.pallas import tpu as pltpu
from jax.experimental.pallas import tpu_sc as plsc

N_NODES = 10000
N_EDGES = 160000
N_ANGLES = 400000
F = 128
L_MAX = 4
N_MAX = 4
CUTOFF = 5.0
CUT3 = 4.0
NUM_EL = 108
NBLOCKS = 4

R = 256
EP = 160512
AP = 400384
NP = 10240
GE = EP // R
GA = AP // R
GN = NP // R

_PREC = jax.lax.Precision.HIGHEST


def _swish(x):
    return x / (1.0 + jnp.exp(-x))


def _poly_cutoff(r, c):
    t = jnp.clip(r / c, 0.0, 1.0)
    return 1.0 - 6.0 * t ** 5 + 15.0 * t ** 4 - 10.0 * t ** 3


def _bessel_list(r, cutoff):
    r_ = r + 1e-8
    s = np.sqrt(2.0 / cutoff).astype(np.float32)
    return [s * jnp.sin((n + 1) * np.float32(np.pi) * r_ / cutoff) / r_
            for n in range(N_MAX + 1)]


def _legendre_list(c):
    polys = [jnp.ones_like(c), c]
    for l in range(2, L_MAX + 1):
        polys.append(((2 * l - 1) * c * polys[-1] - (l - 1) * polys[-2]) / l)
    return polys


def _k_bas_edge(dist3d):
    def body(r_ref, *outs):
        r = r_ref[...]
        e0 = _bessel_list(r, CUTOFF)
        fc = _poly_cutoff(r, CUTOFF)
        for n in range(N_MAX + 1):
            outs[n][...] = e0[n]
            outs[5 + n][...] = e0[n] * fc

    return pl.pallas_call(
        body,
        grid=(GE,),
        in_specs=[pl.BlockSpec((1, 2, 128), lambda i: (i, 0, 0))],
        out_specs=[pl.BlockSpec((1, 2, 128), lambda i: (i, 0, 0))] * 10,
        out_shape=[jax.ShapeDtypeStruct((GE, 2, 128), jnp.float32)] * 10,
    )(dist3d)


def _k_bas_ang(norm3d, cos3d):
    def body(r_ref, c_ref, *outs):
        r = r_ref[...]
        c = c_ref[...]
        rad = _bessel_list(r, CUT3)
        leg = _legendre_list(c)
        fc3 = _poly_cutoff(r, CUT3)
        for n in range(N_MAX + 1):
            outs[n][...] = rad[n] * fc3
            outs[5 + n][...] = leg[n]

    return pl.pallas_call(
        body,
        grid=(GA,),
        in_specs=[pl.BlockSpec((1, 2, 128), lambda i: (i, 0, 0))] * 2,
        out_specs=[pl.BlockSpec((1, 2, 128), lambda i: (i, 0, 0))] * 10,
        out_shape=[jax.ShapeDtypeStruct((GA, 2, 128), jnp.float32)] * 10,
    )(norm3d, cos3d)


def _k_enc(E0s, enc_Wp, enc_b, We3_0):
    def body(e0_ref, w_ref, b_ref, w3_ref, e_ref, t_ref):
        acc = jnp.dot(e0_ref[...], w_ref[...], precision=_PREC) + b_ref[...]
        e = _swish(acc)
        e_ref[...] = e
        t_ref[...] = _swish(jnp.dot(e, w3_ref[...], precision=_PREC))

    return pl.pallas_call(
        body,
        grid=(GE,),
        in_specs=[pl.BlockSpec((R, 16), lambda i: (i, 0)),
                  pl.BlockSpec((16, F), lambda i: (0, 0)),
                  pl.BlockSpec((1, F), lambda i: (0, 0)),
                  pl.BlockSpec((F, F), lambda i: (0, 0))],
        out_specs=[pl.BlockSpec((R, F), lambda i: (i, 0)),
                   pl.BlockSpec((R, F), lambda i: (i, 0))],
        out_shape=[jax.ShapeDtypeStruct((EP, F), jnp.float32),
                   jax.ShapeDtypeStruct((EP, F), jnp.float32)],
    )(E0s, enc_Wp, enc_b, We3_0)


def _k_emb(atomic_col, emb_pad):
    def body(a_ref, w_ref, o_ref):
        a = a_ref[...]
        lanes = lax.broadcasted_iota(jnp.int32, (1, F), 1)
        oh = (a == lanes).astype(jnp.float32)
        o_ref[...] = jnp.dot(oh, w_ref[...], precision=_PREC)

    return pl.pallas_call(
        body,
        grid=(GN,),
        in_specs=[pl.BlockSpec((R, 1), lambda i: (i, 0)),
                  pl.BlockSpec((F, F), lambda i: (0, 0))],
        out_specs=pl.BlockSpec((R, F), lambda i: (i, 0)),
        out_shape=jax.ShapeDtypeStruct((NP, F), jnp.float32),
    )(atomic_col, emb_pad)


def _k_msg3_cumsum(P, g, SA, SB, Wang_pad, Ltri):
    def body(p_ref, g_ref, sa_ref, sb_ref, w_ref, l_ref, c_ref, carry):
        i = pl.program_id(0)

        @pl.when(i == 0)
        def _():
            carry[...] = jnp.zeros((8, F), jnp.float32)

        p = p_ref[...]
        ang = (jnp.dot(p, sa_ref[...], precision=_PREC)
               * jnp.dot(p, sb_ref[...], precision=_PREC))
        a = jnp.dot(ang, w_ref[...], precision=_PREC)
        msg = a * g_ref[...]
        cv = carry[0:1, :]
        c_ref[...] = cv + jnp.dot(l_ref[...], msg, precision=_PREC)
        carry[0:1, :] = cv + jnp.sum(msg, axis=0, keepdims=True)

    return pl.pallas_call(
        body,
        grid=(GA,),
        in_specs=[pl.BlockSpec((R, 16), lambda i: (i, 0)),
                  pl.BlockSpec((R, F), lambda i: (i, 0)),
                  pl.BlockSpec((16, 32), lambda i: (0, 0)),
                  pl.BlockSpec((16, 32), lambda i: (0, 0)),
                  pl.BlockSpec((32, F), lambda i: (0, 0)),
                  pl.BlockSpec((R, R), lambda i: (0, 0))],
        out_specs=pl.BlockSpec((R, F), lambda i: (i, 0)),
        out_shape=jax.ShapeDtypeStruct((AP, F), jnp.float32),
        scratch_shapes=[pltpu.VMEM((8, F), jnp.float32)],
    )(P, g, SA, SB, Wang_pad, Ltri)


def _k_edge_node(Ga, Gb, e, xs, xd, E0s, W3o, Wedge, Wnode, WegP, WngP,
                 Ltri, We3n, emit_t):
    def body(ga_ref, gb_ref, e_ref, xs_ref, xd_ref, e0_ref, w3o_ref, we_ref,
             wn_ref, weg_ref, wng_ref, l_ref, we3_ref, *out_and_scratch):
        if emit_t:
            e2_ref, c_ref, t_ref, carry = out_and_scratch
        else:
            e2_ref, c_ref, carry = out_and_scratch
        i = pl.program_id(0)

        @pl.when(i == 0)
        def _():
            carry[...] = jnp.zeros((8, F), jnp.float32)

        agg3 = gb_ref[...] - ga_ref[...]
        e1 = e_ref[...] + _swish(jnp.dot(agg3, w3o_ref[...], precision=_PREC))

        e0 = e0_ref[...]
        gate_e = jnp.dot(e0, weg_ref[...], precision=_PREC)
        gate_n = jnp.dot(e0, wng_ref[...], precision=_PREC)

        xs = xs_ref[...]
        xd = xd_ref[...]
        we = we_ref[...]
        wn = wn_ref[...]
        arg_e = (jnp.dot(xs, we[0:F, :], precision=_PREC)
                 + jnp.dot(xd, we[F:2 * F, :], precision=_PREC)
                 + jnp.dot(e1, we[2 * F:3 * F, :], precision=_PREC))
        e2 = e1 + _swish(arg_e) * gate_e
        arg_n = (jnp.dot(xs, wn[0:F, :], precision=_PREC)
                 + jnp.dot(xd, wn[F:2 * F, :], precision=_PREC)
                 + jnp.dot(e1, wn[2 * F:3 * F, :], precision=_PREC))
        msg = _swish(arg_n) * gate_n

        cv = carry[0:1, :]
        c_ref[...] = cv + jnp.dot(l_ref[...], msg, precision=_PREC)
        carry[0:1, :] = cv + jnp.sum(msg, axis=0, keepdims=True)
        e2_ref[...] = e2
        if emit_t:
            t_ref[...] = _swish(jnp.dot(e2, we3_ref[...], precision=_PREC))

    n_out = 3 if emit_t else 2
    return pl.pallas_call(
        body,
        grid=(GE,),
        in_specs=[pl.BlockSpec((R, F), lambda i: (i, 0)),
                  pl.BlockSpec((R, F), lambda i: (i, 0)),
                  pl.BlockSpec((R, F), lambda i: (i, 0)),
                  pl.BlockSpec((R, F), lambda i: (i, 0)),
                  pl.BlockSpec((R, F), lambda i: (i, 0)),
                  pl.BlockSpec((R, 16), lambda i: (i, 0)),
                  pl.BlockSpec((F, F), lambda i: (0, 0)),
                  pl.BlockSpec((3 * F, F), lambda i: (0, 0)),
                  pl.BlockSpec((3 * F, F), lambda i: (0, 0)),
                  pl.BlockSpec((16, F), lambda i: (0, 0)),
                  pl.BlockSpec((16, F), lambda i: (0, 0)),
                  pl.BlockSpec((R, R), lambda i: (0, 0)),
                  pl.BlockSpec((F, F), lambda i: (0, 0))],
        out_specs=[pl.BlockSpec((R, F), lambda i: (i, 0))] * n_out,
        out_shape=[jax.ShapeDtypeStruct((EP, F), jnp.float32)] * n_out,
        scratch_shapes=[pltpu.VMEM((8, F), jnp.float32)],
    )(Ga, Gb, e, xs, xd, E0s, W3o, Wedge, Wnode, WegP, WngP, Ltri, We3n)


def _k_xupd(x, Pa, Pb):
    def body(x_ref, a_ref, b_ref, o_ref):
        o_ref[...] = x_ref[...] + b_ref[...] - a_ref[...]

    return pl.pallas_call(
        body,
        grid=(GN,),
        in_specs=[pl.BlockSpec((R, F), lambda i: (i, 0))] * 3,
        out_specs=pl.BlockSpec((R, F), lambda i: (i, 0)),
        out_shape=jax.ShapeDtypeStruct((NP, F), jnp.float32),
    )(x, Pa, Pb)


def _k_out(x, eW1, eb1, eW2, eb2, eW3_row):
    def body(x_ref, w1_ref, b1_ref, w2_ref, b2_ref, w3_ref, o_ref):
        h = _swish(jnp.dot(x_ref[...], w1_ref[...], precision=_PREC)
                   + b1_ref[...])
        h = _swish(jnp.dot(h, w2_ref[...], precision=_PREC) + b2_ref[...])
        o_ref[...] = jnp.sum(h * w3_ref[...], axis=1, keepdims=True)

    return pl.pallas_call(
        body,
        grid=(GN,),
        in_specs=[pl.BlockSpec((R, F), lambda i: (i, 0)),
                  pl.BlockSpec((F, F), lambda i: (0, 0)),
                  pl.BlockSpec((1, F), lambda i: (0, 0)),
                  pl.BlockSpec((F, F), lambda i: (0, 0)),
                  pl.BlockSpec((1, F), lambda i: (0, 0)),
                  pl.BlockSpec((1, F), lambda i: (0, 0))],
        out_specs=pl.BlockSpec((R, 1), lambda i: (i, 0)),
        out_shape=jax.ShapeDtypeStruct((NP, 1), jnp.float32),
    )(x, eW1, eb1, eW2, eb2, eW3_row)


_NW = 32
_CH = 128


def _gather_one_call(table, idx, width=F):
    B = idx.shape[0]
    per = B // _NW
    halfA = ((per // 2) // 8) * 8
    lenB = per - halfA
    nf = min(halfA // _CH, lenB // _CH)

    def _tail_chunks(start, length):
        out = []
        done = nf * _CH
        while done < length:
            n = min(_CH, length - done)
            out.append((start + done, n))
            done += n
        return out

    mesh = plsc.VectorSubcoreMesh(core_axis_name="c", subcore_axis_name="s")

    @functools.partial(
        pl.kernel, mesh=mesh,
        out_type=jax.ShapeDtypeStruct((B, width), jnp.float32),
        scratch_types=[pltpu.VMEM((_CH,), jnp.int32),
                       pltpu.VMEM((_CH, width), jnp.float32),
                       pltpu.VMEM((_CH,), jnp.int32),
                       pltpu.VMEM((_CH, width), jnp.float32),
                       pltpu.SemaphoreType.DMA,
                       pltpu.SemaphoreType.DMA],
    )
    def k(tab, ih, oh, iva, rva, ivb, rvb, sa, sb):
        wid = lax.axis_index("s") * 2 + lax.axis_index("c")
        base = wid * per

        def do1(off, n, iv, rv, sem):
            pltpu.sync_copy(ih.at[pl.ds(off, n)], iv.at[pl.ds(0, n)])
            pltpu.async_copy(tab.at[iv.at[pl.ds(0, n)]],
                             rv.at[pl.ds(0, n)], sem).wait()
            pltpu.sync_copy(rv.at[pl.ds(0, n)], oh.at[pl.ds(off, n)])

        def do(offa, offb, n):
            pltpu.sync_copy(ih.at[pl.ds(offa, n)], iva.at[pl.ds(0, n)])
            cpa = pltpu.async_copy(tab.at[iva.at[pl.ds(0, n)]],
                                   rva.at[pl.ds(0, n)], sa)
            pltpu.sync_copy(ih.at[pl.ds(offb, n)], ivb.at[pl.ds(0, n)])
            cpb = pltpu.async_copy(tab.at[ivb.at[pl.ds(0, n)]],
                                   rvb.at[pl.ds(0, n)], sb)
            cpa.wait()
            pltpu.sync_copy(rva.at[pl.ds(0, n)], oh.at[pl.ds(offa, n)])
            cpb.wait()
            pltpu.sync_copy(rvb.at[pl.ds(0, n)], oh.at[pl.ds(offb, n)])

        def body(i, _):
            do(base + i * _CH, base + halfA + i * _CH, _CH)
            return ()

        lax.fori_loop(0, nf, body, ())
        for off, n in _tail_chunks(base, halfA):
            do1(off, n, iva, rva, sa)
        for off, n in _tail_chunks(base + halfA, lenB):
            do1(off, n, ivb, rvb, sb)

    return k(table, idx)


def _gather_pair_call(table, idx_a, idx_b):
    B = idx_a.shape[0]
    per = B // _NW
    nfull = per // _CH
    rem = per - nfull * _CH
    mesh = plsc.VectorSubcoreMesh(core_axis_name="c", subcore_axis_name="s")

    @functools.partial(
        pl.kernel, mesh=mesh,
        out_type=(jax.ShapeDtypeStruct((B, F), jnp.float32),
                  jax.ShapeDtypeStruct((B, F), jnp.float32)),
        scratch_types=[pltpu.VMEM((_CH,), jnp.int32),
                       pltpu.VMEM((_CH, F), jnp.float32),
                       pltpu.VMEM((_CH,), jnp.int32),
                       pltpu.VMEM((_CH, F), jnp.float32),
                       pltpu.SemaphoreType.DMA,
                       pltpu.SemaphoreType.DMA],
    )
    def k(tab, ia, ib, oa, ob, iva, rva, ivb, rvb, sa, sb):
        wid = lax.axis_index("s") * 2 + lax.axis_index("c")
        base = wid * per

        def do(off, n):
            pltpu.sync_copy(ia.at[pl.ds(off, n)], iva.at[pl.ds(0, n)])
            cpa = pltpu.async_copy(tab.at[iva.at[pl.ds(0, n)]],
                                   rva.at[pl.ds(0, n)], sa)
            pltpu.sync_copy(ib.at[pl.ds(off, n)], ivb.at[pl.ds(0, n)])
            cpb = pltpu.async_copy(tab.at[ivb.at[pl.ds(0, n)]],
                                   rvb.at[pl.ds(0, n)], sb)
            cpa.wait()
            pltpu.sync_copy(rva.at[pl.ds(0, n)], oa.at[pl.ds(off, n)])
            cpb.wait()
            pltpu.sync_copy(rvb.at[pl.ds(0, n)], ob.at[pl.ds(off, n)])

        def body(i, _):
            do(base + i * _CH, _CH)
            return ()

        lax.fori_loop(0, nfull, body, ())
        if rem:
            do(base + nfull * _CH, rem)

    return k(table, idx_a, idx_b)


def _pad1(a, n, val):
    return jnp.concatenate(
        [a, jnp.full((n - a.shape[0],), val, a.dtype)])


def kernel(atomic_numbers, edge_index, edge_dist, three_body_indices, norm_ik,
           three_body_cos_angles, total_num_bonds, total_num_angles, params):
    p = params
    f32 = jnp.float32
    i32 = jnp.int32
    tbi0 = three_body_indices[:, 0].astype(i32)
    tbi1 = three_body_indices[:, 1].astype(i32)
    src = edge_index[0].astype(i32)
    dst = edge_index[1].astype(i32)

    eperm = jnp.argsort(dst).astype(i32)
    inv_eperm = jnp.zeros((N_EDGES,), i32).at[eperm].set(
        jnp.arange(N_EDGES, dtype=i32))
    tbi0r = inv_eperm[tbi0]
    tbi1r = inv_eperm[tbi1]
    aperm = jnp.argsort(tbi0r).astype(i32)

    cnt_a = jnp.zeros((N_EDGES,), i32).at[tbi0r].add(1)
    csa = jnp.cumsum(cnt_a)
    rsA_a = _pad1(jnp.concatenate([jnp.zeros((1,), i32), csa[:-1]]),
                  EP, N_ANGLES)
    rsB_a = _pad1(csa, EP, N_ANGLES)
    cnt_n = jnp.zeros((N_NODES,), i32).at[dst].add(1)
    csn = jnp.cumsum(cnt_n)
    rsA_n = _pad1(jnp.concatenate([jnp.zeros((1,), i32), csn[:-1]]),
                  NP, N_EDGES)
    rsB_n = _pad1(csn, NP, N_EDGES)

    eperm_p = _pad1(eperm, EP, 0)
    aperm_p = _pad1(aperm, AP, 0)

    dist3d = _pad1(edge_dist.astype(f32), EP, 10.0).reshape(GE, 2, 128)
    norm3d = _pad1(norm_ik.astype(f32), AP, 10.0).reshape(GA, 2, 128)
    cos3d = _pad1(three_body_cos_angles.astype(f32), AP, 0.0).reshape(
        GA, 2, 128)

    eb = _k_bas_edge(dist3d)
    ab = _k_bas_ang(norm3d, cos3d)

    Etab = jnp.concatenate(
        [jnp.stack(
            [o.reshape(EP) for o in eb]
            + [lax.bitcast_convert_type(_pad1(src, EP, 0), f32),
               lax.bitcast_convert_type(_pad1(dst, EP, 0), f32)], axis=1),
         jnp.zeros((EP, 116), f32)], axis=1)
    Atab = jnp.concatenate(
        [jnp.stack(
            [o.reshape(AP) for o in ab]
            + [lax.bitcast_convert_type(_pad1(tbi1r, AP, 0), f32)], axis=1),
         jnp.zeros((AP, 117), f32)], axis=1)

    E0w = _gather_one_call(Etab, eperm_p)
    Pw = _gather_one_call(Atab, aperm_p)
    E0s = E0w[:, :16]
    Ps = Pw[:, :16]

    src_p = lax.bitcast_convert_type(E0w[:, 10], i32)
    dst_p = lax.bitcast_convert_type(E0w[:, 11], i32)
    tbi1_p = lax.bitcast_convert_type(Pw[:, 10], i32)

    emb_pad = jnp.zeros((F, F), f32).at[:NUM_EL].set(p["emb"].astype(f32))
    enc_Wp = jnp.zeros((16, F), f32).at[:N_MAX + 1].set(p["enc_W"].astype(f32))
    enc_b = p["enc_b"].astype(f32)[None, :]
    Ltri = jnp.asarray(np.tril(np.ones((R, R), np.float32), -1))
    sa_np = np.zeros((16, 32), np.float32)
    sb_np = np.zeros((16, 32), np.float32)
    for l in range(L_MAX + 1):
        for n in range(N_MAX + 1):
            sa_np[n, l * 5 + n] = 1.0
            sb_np[5 + l, l * 5 + n] = 1.0
    SA = jnp.asarray(sa_np)
    SB = jnp.asarray(sb_np)

    blocks = p["blocks"]
    Wang_pads = [jnp.zeros((32, F), f32).at[:25].set(b["Wang"].astype(f32))
                 for b in blocks]
    WegPs = [jnp.zeros((16, F), f32).at[5:10].set(b["Weg"].astype(f32))
             for b in blocks]
    WngPs = [jnp.zeros((16, F), f32).at[5:10].set(b["Wng"].astype(f32))
             for b in blocks]

    atomic_col = _pad1(atomic_numbers.astype(i32), NP, 0)[:, None]
    x = _k_emb(atomic_col, emb_pad)
    e, t = _k_enc(E0s, enc_Wp, enc_b, blocks[0]["We3"].astype(f32))

    for b in range(NBLOCKS):
        blk = blocks[b]
        g = _gather_one_call(t, tbi1_p)
        C = _k_msg3_cumsum(Ps, g, SA, SB, Wang_pads[b], Ltri)
        Ga, Gb = _gather_pair_call(C, rsA_a, rsB_a)
        xs, xd = _gather_pair_call(x, src_p, dst_p)
        emit_t = b < NBLOCKS - 1
        We3n = (blocks[b + 1]["We3"] if emit_t else blocks[0]["We3"]).astype(f32)
        outs = _k_edge_node(Ga, Gb, e, xs, xd, E0s, blk["W3o"].astype(f32),
                            blk["Wedge"].astype(f32), blk["Wnode"].astype(f32),
                            WegPs[b], WngPs[b], Ltri, We3n, emit_t)
        if emit_t:
            e, Cmsg, t = outs
        else:
            e, Cmsg = outs
        Pa, Pb = _gather_pair_call(Cmsg, rsA_n, rsB_n)
        x = _k_xupd(x, Pa, Pb)

    energy = _k_out(x, p["eW1"].astype(f32), p["eb1"].astype(f32)[None, :],
                    p["eW2"].astype(f32), p["eb2"].astype(f32)[None, :],
                    p["eW3"].astype(f32)[:, 0][None, :])
    return energy[:N_NODES] + p["eb3"].astype(f32)[None, :]


# speedup vs baseline: 1.5084x; 1.2105x over previous
import functools

import jax
import jax.numpy as jnp
import numpy as np
from jax import lax
from jax.experimental import pallas as pl
from jax.experimental.pallas import tpu as pltpu
from jax.experimental.pallas import tpu_sc as plsc

N_NODES = 10000
N_EDGES = 160000
N_ANGLES = 400000
F = 128
L_MAX = 4
N_MAX = 4
CUTOFF = 5.0
CUT3 = 4.0
NUM_EL = 108
NBLOCKS = 4

R = 256
EP = 160512
AP = 400384
NP = 10240
GE = EP // R
GA = AP // R
GN = NP // R

_PREC = jax.lax.Precision.DEFAULT


def _swish(x):
    return x * (0.5 * jnp.tanh(0.5 * x) + 0.5)


def _poly_cutoff(r, c):
    t = jnp.clip(r / c, 0.0, 1.0)
    return 1.0 - 6.0 * t ** 5 + 15.0 * t ** 4 - 10.0 * t ** 3


def _bessel_list(r, cutoff):
    r_ = r + 1e-8
    s = np.sqrt(2.0 / cutoff).astype(np.float32)
    return [s * jnp.sin((n + 1) * np.float32(np.pi) * r_ / cutoff) / r_
            for n in range(N_MAX + 1)]


def _legendre_list(c):
    polys = [jnp.ones_like(c), c]
    for l in range(2, L_MAX + 1):
        polys.append(((2 * l - 1) * c * polys[-1] - (l - 1) * polys[-2]) / l)
    return polys


def _k_bas_edge(dist3d):
    def body(r_ref, *outs):
        r = r_ref[...]
        e0 = _bessel_list(r, CUTOFF)
        fc = _poly_cutoff(r, CUTOFF)
        for n in range(N_MAX + 1):
            outs[n][...] = e0[n]
            outs[5 + n][...] = e0[n] * fc

    return pl.pallas_call(
        body,
        grid=(GE,),
        in_specs=[pl.BlockSpec((1, 2, 128), lambda i: (i, 0, 0))],
        out_specs=[pl.BlockSpec((1, 2, 128), lambda i: (i, 0, 0))] * 10,
        out_shape=[jax.ShapeDtypeStruct((GE, 2, 128), jnp.float32)] * 10,
    )(dist3d)


def _k_bas_ang(norm3d, cos3d):
    def body(r_ref, c_ref, *outs):
        r = r_ref[...]
        c = c_ref[...]
        rad = _bessel_list(r, CUT3)
        leg = _legendre_list(c)
        fc3 = _poly_cutoff(r, CUT3)
        for n in range(N_MAX + 1):
            outs[n][...] = rad[n] * fc3
            outs[5 + n][...] = leg[n]

    return pl.pallas_call(
        body,
        grid=(GA,),
        in_specs=[pl.BlockSpec((1, 2, 128), lambda i: (i, 0, 0))] * 2,
        out_specs=[pl.BlockSpec((1, 2, 128), lambda i: (i, 0, 0))] * 10,
        out_shape=[jax.ShapeDtypeStruct((GA, 2, 128), jnp.float32)] * 10,
    )(norm3d, cos3d)


def _k_enc(E0s, enc_Wp, enc_b, We3_0):
    def body(e0_ref, w_ref, b_ref, w3_ref, e_ref, t_ref):
        acc = jnp.dot(e0_ref[...], w_ref[...], precision=_PREC) + b_ref[...]
        e = _swish(acc)
        e_ref[...] = e
        t_ref[...] = _swish(jnp.dot(e, w3_ref[...], precision=_PREC))

    return pl.pallas_call(
        body,
        grid=(GE,),
        in_specs=[pl.BlockSpec((R, 16), lambda i: (i, 0)),
                  pl.BlockSpec((16, F), lambda i: (0, 0)),
                  pl.BlockSpec((1, F), lambda i: (0, 0)),
                  pl.BlockSpec((F, F), lambda i: (0, 0))],
        out_specs=[pl.BlockSpec((R, F), lambda i: (i, 0)),
                   pl.BlockSpec((R, F), lambda i: (i, 0))],
        out_shape=[jax.ShapeDtypeStruct((EP, F), jnp.float32),
                   jax.ShapeDtypeStruct((EP, F), jnp.float32)],
    )(E0s, enc_Wp, enc_b, We3_0)


def _k_emb(atomic_col, emb_pad):
    def body(a_ref, w_ref, o_ref):
        a = a_ref[...]
        lanes = lax.broadcasted_iota(jnp.int32, (1, F), 1)
        oh = (a == lanes).astype(jnp.float32)
        o_ref[...] = jnp.dot(oh, w_ref[...], precision=_PREC)

    return pl.pallas_call(
        body,
        grid=(GN,),
        in_specs=[pl.BlockSpec((R, 1), lambda i: (i, 0)),
                  pl.BlockSpec((F, F), lambda i: (0, 0))],
        out_specs=pl.BlockSpec((R, F), lambda i: (i, 0)),
        out_shape=jax.ShapeDtypeStruct((NP, F), jnp.float32),
    )(atomic_col, emb_pad)


def _k_msg3_cumsum(P, g, SA, SB, Wang_pad, Ltri):
    def body(p_ref, g_ref, sa_ref, sb_ref, w_ref, l_ref, c_ref, carry):
        i = pl.program_id(0)

        @pl.when(i == 0)
        def _():
            carry[...] = jnp.zeros((8, F), jnp.float32)

        p = p_ref[...]
        ang = (jnp.dot(p, sa_ref[...], precision=_PREC)
               * jnp.dot(p, sb_ref[...], precision=_PREC))
        a = jnp.dot(ang, w_ref[...], precision=_PREC)
        msg = a * g_ref[...]
        cv = carry[0:1, :]
        c_ref[...] = cv + jnp.dot(l_ref[...], msg, precision=_PREC)
        carry[0:1, :] = cv + jnp.sum(msg, axis=0, keepdims=True)

    return pl.pallas_call(
        body,
        grid=(GA,),
        in_specs=[pl.BlockSpec((R, 16), lambda i: (i, 0)),
                  pl.BlockSpec((R, F), lambda i: (i, 0)),
                  pl.BlockSpec((16, 32), lambda i: (0, 0)),
                  pl.BlockSpec((16, 32), lambda i: (0, 0)),
                  pl.BlockSpec((32, F), lambda i: (0, 0)),
                  pl.BlockSpec((R, R), lambda i: (0, 0))],
        out_specs=pl.BlockSpec((R, F), lambda i: (i, 0)),
        out_shape=jax.ShapeDtypeStruct((AP, F), jnp.float32),
        scratch_shapes=[pltpu.VMEM((8, F), jnp.float32)],
    )(P, g, SA, SB, Wang_pad, Ltri)


def _k_edge_node(Ga, Gb, e, xs, xd, E0s, W3o, Wedge, Wnode, WegP, WngP,
                 Ltri, We3n, emit_t):
    def body(ga_ref, gb_ref, e_ref, xs_ref, xd_ref, e0_ref, w3o_ref, we_ref,
             wn_ref, weg_ref, wng_ref, l_ref, we3_ref, *out_and_scratch):
        if emit_t:
            e2_ref, c_ref, t_ref, carry = out_and_scratch
        else:
            e2_ref, c_ref, carry = out_and_scratch
        i = pl.program_id(0)

        @pl.when(i == 0)
        def _():
            carry[...] = jnp.zeros((8, F), jnp.float32)

        agg3 = gb_ref[...] - ga_ref[...]
        e1 = e_ref[...] + _swish(jnp.dot(agg3, w3o_ref[...], precision=_PREC))

        e0 = e0_ref[...]
        gate_e = jnp.dot(e0, weg_ref[...], precision=_PREC)
        gate_n = jnp.dot(e0, wng_ref[...], precision=_PREC)

        xs = xs_ref[...]
        xd = xd_ref[...]
        we = we_ref[...]
        wn = wn_ref[...]
        arg_e = (jnp.dot(xs, we[0:F, :], precision=_PREC)
                 + jnp.dot(xd, we[F:2 * F, :], precision=_PREC)
                 + jnp.dot(e1, we[2 * F:3 * F, :], precision=_PREC))
        e2 = e1 + _swish(arg_e) * gate_e
        arg_n = (jnp.dot(xs, wn[0:F, :], precision=_PREC)
                 + jnp.dot(xd, wn[F:2 * F, :], precision=_PREC)
                 + jnp.dot(e1, wn[2 * F:3 * F, :], precision=_PREC))
        msg = _swish(arg_n) * gate_n

        cv = carry[0:1, :]
        c_ref[...] = cv + jnp.dot(l_ref[...], msg, precision=_PREC)
        carry[0:1, :] = cv + jnp.sum(msg, axis=0, keepdims=True)
        e2_ref[...] = e2
        if emit_t:
            t_ref[...] = _swish(jnp.dot(e2, we3_ref[...], precision=_PREC))

    n_out = 3 if emit_t else 2
    return pl.pallas_call(
        body,
        grid=(GE,),
        in_specs=[pl.BlockSpec((R, F), lambda i: (i, 0)),
                  pl.BlockSpec((R, F), lambda i: (i, 0)),
                  pl.BlockSpec((R, F), lambda i: (i, 0)),
                  pl.BlockSpec((R, F), lambda i: (i, 0)),
                  pl.BlockSpec((R, F), lambda i: (i, 0)),
                  pl.BlockSpec((R, 16), lambda i: (i, 0)),
                  pl.BlockSpec((F, F), lambda i: (0, 0)),
                  pl.BlockSpec((3 * F, F), lambda i: (0, 0)),
                  pl.BlockSpec((3 * F, F), lambda i: (0, 0)),
                  pl.BlockSpec((16, F), lambda i: (0, 0)),
                  pl.BlockSpec((16, F), lambda i: (0, 0)),
                  pl.BlockSpec((R, R), lambda i: (0, 0)),
                  pl.BlockSpec((F, F), lambda i: (0, 0))],
        out_specs=[pl.BlockSpec((R, F), lambda i: (i, 0))] * n_out,
        out_shape=[jax.ShapeDtypeStruct((EP, F), jnp.float32)] * n_out,
        scratch_shapes=[pltpu.VMEM((8, F), jnp.float32)],
    )(Ga, Gb, e, xs, xd, E0s, W3o, Wedge, Wnode, WegP, WngP, Ltri, We3n)


def _k_xupd(x, Pa, Pb):
    def body(x_ref, a_ref, b_ref, o_ref):
        o_ref[...] = x_ref[...] + b_ref[...] - a_ref[...]

    return pl.pallas_call(
        body,
        grid=(GN,),
        in_specs=[pl.BlockSpec((R, F), lambda i: (i, 0))] * 3,
        out_specs=pl.BlockSpec((R, F), lambda i: (i, 0)),
        out_shape=jax.ShapeDtypeStruct((NP, F), jnp.float32),
    )(x, Pa, Pb)


def _k_out(x, eW1, eb1, eW2, eb2, eW3_row):
    def body(x_ref, w1_ref, b1_ref, w2_ref, b2_ref, w3_ref, o_ref):
        h = _swish(jnp.dot(x_ref[...], w1_ref[...], precision=_PREC)
                   + b1_ref[...])
        h = _swish(jnp.dot(h, w2_ref[...], precision=_PREC) + b2_ref[...])
        o_ref[...] = jnp.sum(h * w3_ref[...], axis=1, keepdims=True)

    return pl.pallas_call(
        body,
        grid=(GN,),
        in_specs=[pl.BlockSpec((R, F), lambda i: (i, 0)),
                  pl.BlockSpec((F, F), lambda i: (0, 0)),
                  pl.BlockSpec((1, F), lambda i: (0, 0)),
                  pl.BlockSpec((F, F), lambda i: (0, 0)),
                  pl.BlockSpec((1, F), lambda i: (0, 0)),
                  pl.BlockSpec((1, F), lambda i: (0, 0))],
        out_specs=pl.BlockSpec((R, 1), lambda i: (i, 0)),
        out_shape=jax.ShapeDtypeStruct((NP, 1), jnp.float32),
    )(x, eW1, eb1, eW2, eb2, eW3_row)


_NW = 32
_CH = 128


def _gather_one_call(table, idx, width=F):
    B = idx.shape[0]
    per = B // _NW
    halfA = ((per // 2) // 8) * 8
    lenB = per - halfA
    nf = min(halfA // _CH, lenB // _CH)

    def _tail_chunks(start, length):
        out = []
        done = nf * _CH
        while done < length:
            n = min(_CH, length - done)
            out.append((start + done, n))
            done += n
        return out

    mesh = plsc.VectorSubcoreMesh(core_axis_name="c", subcore_axis_name="s")

    @functools.partial(
        pl.kernel, mesh=mesh,
        out_type=jax.ShapeDtypeStruct((B, width), jnp.float32),
        scratch_types=[pltpu.VMEM((_CH,), jnp.int32),
                       pltpu.VMEM((_CH, width), jnp.float32),
                       pltpu.VMEM((_CH,), jnp.int32),
                       pltpu.VMEM((_CH, width), jnp.float32),
                       pltpu.SemaphoreType.DMA,
                       pltpu.SemaphoreType.DMA],
    )
    def k(tab, ih, oh, iva, rva, ivb, rvb, sa, sb):
        wid = lax.axis_index("s") * 2 + lax.axis_index("c")
        base = wid * per

        def do1(off, n, iv, rv, sem):
            pltpu.sync_copy(ih.at[pl.ds(off, n)], iv.at[pl.ds(0, n)])
            pltpu.async_copy(tab.at[iv.at[pl.ds(0, n)]],
                             rv.at[pl.ds(0, n)], sem).wait()
            pltpu.sync_copy(rv.at[pl.ds(0, n)], oh.at[pl.ds(off, n)])

        def do(offa, offb, n):
            pltpu.sync_copy(ih.at[pl.ds(offa, n)], iva.at[pl.ds(0, n)])
            cpa = pltpu.async_copy(tab.at[iva.at[pl.ds(0, n)]],
                                   rva.at[pl.ds(0, n)], sa)
            pltpu.sync_copy(ih.at[pl.ds(offb, n)], ivb.at[pl.ds(0, n)])
            cpb = pltpu.async_copy(tab.at[ivb.at[pl.ds(0, n)]],
                                   rvb.at[pl.ds(0, n)], sb)
            cpa.wait()
            pltpu.sync_copy(rva.at[pl.ds(0, n)], oh.at[pl.ds(offa, n)])
            cpb.wait()
            pltpu.sync_copy(rvb.at[pl.ds(0, n)], oh.at[pl.ds(offb, n)])

        def body(i, _):
            do(base + i * _CH, base + halfA + i * _CH, _CH)
            return ()

        lax.fori_loop(0, nf, body, ())
        for off, n in _tail_chunks(base, halfA):
            do1(off, n, iva, rva, sa)
        for off, n in _tail_chunks(base + halfA, lenB):
            do1(off, n, ivb, rvb, sb)

    return k(table, idx)


def _gather_pair_call(table, idx_a, idx_b):
    B = idx_a.shape[0]
    per = B // _NW
    nfull = per // _CH
    rem = per - nfull * _CH
    mesh = plsc.VectorSubcoreMesh(core_axis_name="c", subcore_axis_name="s")

    @functools.partial(
        pl.kernel, mesh=mesh,
        out_type=(jax.ShapeDtypeStruct((B, F), jnp.float32),
                  jax.ShapeDtypeStruct((B, F), jnp.float32)),
        scratch_types=[pltpu.VMEM((_CH,), jnp.int32),
                       pltpu.VMEM((_CH, F), jnp.float32),
                       pltpu.VMEM((_CH,), jnp.int32),
                       pltpu.VMEM((_CH, F), jnp.float32),
                       pltpu.SemaphoreType.DMA,
                       pltpu.SemaphoreType.DMA],
    )
    def k(tab, ia, ib, oa, ob, iva, rva, ivb, rvb, sa, sb):
        wid = lax.axis_index("s") * 2 + lax.axis_index("c")
        base = wid * per

        def do(off, n):
            pltpu.sync_copy(ia.at[pl.ds(off, n)], iva.at[pl.ds(0, n)])
            cpa = pltpu.async_copy(tab.at[iva.at[pl.ds(0, n)]],
                                   rva.at[pl.ds(0, n)], sa)
            pltpu.sync_copy(ib.at[pl.ds(off, n)], ivb.at[pl.ds(0, n)])
            cpb = pltpu.async_copy(tab.at[ivb.at[pl.ds(0, n)]],
                                   rvb.at[pl.ds(0, n)], sb)
            cpa.wait()
            pltpu.sync_copy(rva.at[pl.ds(0, n)], oa.at[pl.ds(off, n)])
            cpb.wait()
            pltpu.sync_copy(rvb.at[pl.ds(0, n)], ob.at[pl.ds(off, n)])

        def body(i, _):
            do(base + i * _CH, _CH)
            return ()

        lax.fori_loop(0, nfull, body, ())
        if rem:
            do(base + nfull * _CH, rem)

    return k(table, idx_a, idx_b)


def _pad1(a, n, val):
    return jnp.concatenate(
        [a, jnp.full((n - a.shape[0],), val, a.dtype)])


def kernel(atomic_numbers, edge_index, edge_dist, three_body_indices, norm_ik,
           three_body_cos_angles, total_num_bonds, total_num_angles, params):
    p = params
    f32 = jnp.float32
    i32 = jnp.int32
    tbi0 = three_body_indices[:, 0].astype(i32)
    tbi1 = three_body_indices[:, 1].astype(i32)
    src = edge_index[0].astype(i32)
    dst = edge_index[1].astype(i32)

    eperm = jnp.argsort(dst).astype(i32)
    inv_eperm = jnp.zeros((N_EDGES,), i32).at[eperm].set(
        jnp.arange(N_EDGES, dtype=i32))
    tbi0r = inv_eperm[tbi0]
    tbi1r = inv_eperm[tbi1]
    aperm = jnp.argsort(tbi0r).astype(i32)

    cnt_a = jnp.zeros((N_EDGES,), i32).at[tbi0r].add(1)
    csa = jnp.cumsum(cnt_a)
    rsA_a = _pad1(jnp.concatenate([jnp.zeros((1,), i32), csa[:-1]]),
                  EP, N_ANGLES)
    rsB_a = _pad1(csa, EP, N_ANGLES)
    cnt_n = jnp.zeros((N_NODES,), i32).at[dst].add(1)
    csn = jnp.cumsum(cnt_n)
    rsA_n = _pad1(jnp.concatenate([jnp.zeros((1,), i32), csn[:-1]]),
                  NP, N_EDGES)
    rsB_n = _pad1(csn, NP, N_EDGES)

    eperm_p = _pad1(eperm, EP, 0)
    aperm_p = _pad1(aperm, AP, 0)

    dist3d = _pad1(edge_dist.astype(f32), EP, 10.0).reshape(GE, 2, 128)
    norm3d = _pad1(norm_ik.astype(f32), AP, 10.0).reshape(GA, 2, 128)
    cos3d = _pad1(three_body_cos_angles.astype(f32), AP, 0.0).reshape(
        GA, 2, 128)

    eb = _k_bas_edge(dist3d)
    ab = _k_bas_ang(norm3d, cos3d)

    Etab = jnp.concatenate(
        [jnp.stack(
            [o.reshape(EP) for o in eb]
            + [lax.bitcast_convert_type(_pad1(src, EP, 0), f32),
               lax.bitcast_convert_type(_pad1(dst, EP, 0), f32)], axis=1),
         jnp.zeros((EP, 116), f32)], axis=1)
    Atab = jnp.concatenate(
        [jnp.stack(
            [o.reshape(AP) for o in ab]
            + [lax.bitcast_convert_type(_pad1(tbi1r, AP, 0), f32)], axis=1),
         jnp.zeros((AP, 117), f32)], axis=1)

    E0w = _gather_one_call(Etab, eperm_p)
    Pw = _gather_one_call(Atab, aperm_p)
    E0s = E0w[:, :16]
    Ps = Pw[:, :16]

    src_p = lax.bitcast_convert_type(E0w[:, 10], i32)
    dst_p = lax.bitcast_convert_type(E0w[:, 11], i32)
    tbi1_p = lax.bitcast_convert_type(Pw[:, 10], i32)

    emb_pad = jnp.zeros((F, F), f32).at[:NUM_EL].set(p["emb"].astype(f32))
    enc_Wp = jnp.zeros((16, F), f32).at[:N_MAX + 1].set(p["enc_W"].astype(f32))
    enc_b = p["enc_b"].astype(f32)[None, :]
    Ltri = jnp.asarray(np.tril(np.ones((R, R), np.float32), -1))
    sa_np = np.zeros((16, 32), np.float32)
    sb_np = np.zeros((16, 32), np.float32)
    for l in range(L_MAX + 1):
        for n in range(N_MAX + 1):
            sa_np[n, l * 5 + n] = 1.0
            sb_np[5 + l, l * 5 + n] = 1.0
    SA = jnp.asarray(sa_np)
    SB = jnp.asarray(sb_np)

    blocks = p["blocks"]
    Wang_pads = [jnp.zeros((32, F), f32).at[:25].set(b["Wang"].astype(f32))
                 for b in blocks]
    WegPs = [jnp.zeros((16, F), f32).at[5:10].set(b["Weg"].astype(f32))
             for b in blocks]
    WngPs = [jnp.zeros((16, F), f32).at[5:10].set(b["Wng"].astype(f32))
             for b in blocks]

    atomic_col = _pad1(atomic_numbers.astype(i32), NP, 0)[:, None]
    x = _k_emb(atomic_col, emb_pad)
    e, t = _k_enc(E0s, enc_Wp, enc_b, blocks[0]["We3"].astype(f32))

    for b in range(NBLOCKS):
        blk = blocks[b]
        g = _gather_one_call(t, tbi1_p)
        C = _k_msg3_cumsum(Ps, g, SA, SB, Wang_pads[b], Ltri)
        Ga, Gb = _gather_pair_call(C, rsA_a, rsB_a)
        xs, xd = _gather_pair_call(x, src_p, dst_p)
        emit_t = b < NBLOCKS - 1
        We3n = (blocks[b + 1]["We3"] if emit_t else blocks[0]["We3"]).astype(f32)
        outs = _k_edge_node(Ga, Gb, e, xs, xd, E0s, blk["W3o"].astype(f32),
                            blk["Wedge"].astype(f32), blk["Wnode"].astype(f32),
                            WegPs[b], WngPs[b], Ltri, We3n, emit_t)
        if emit_t:
            e, Cmsg, t = outs
        else:
            e, Cmsg = outs
        Pa, Pb = _gather_pair_call(Cmsg, rsA_n, rsB_n)
        x = _k_xupd(x, Pa, Pb)

    energy = _k_out(x, p["eW1"].astype(f32), p["eb1"].astype(f32)[None, :],
                    p["eW2"].astype(f32), p["eb2"].astype(f32)[None, :],
                    p["eW3"].astype(f32)[:, 0][None, :])
    return energy[:N_NODES] + p["eb3"].astype(f32)[None, :]


# speedup vs baseline: 2.6051x; 1.7270x over previous
import functools

import jax
import jax.numpy as jnp
import numpy as np
from jax import lax
from jax.experimental import pallas as pl
from jax.experimental.pallas import tpu as pltpu
from jax.experimental.pallas import tpu_sc as plsc

N_NODES = 10000
N_EDGES = 160000
N_ANGLES = 400000
F = 128
L_MAX = 4
N_MAX = 4
CUTOFF = 5.0
CUT3 = 4.0
NUM_EL = 108
NBLOCKS = 4

R = 256
EP = 160512
AP = 400384
NP = 10240
GE = EP // R
GA = AP // R
GN = NP // R

_PREC = jax.lax.Precision.DEFAULT


def _swish(x):
    return x * (0.5 * jnp.tanh(0.5 * x) + 0.5)


def _poly_cutoff(r, c):
    t = jnp.clip(r / c, 0.0, 1.0)
    return 1.0 - 6.0 * t ** 5 + 15.0 * t ** 4 - 10.0 * t ** 3


def _bessel_list(r, cutoff):
    r_ = r + 1e-8
    s = np.sqrt(2.0 / cutoff).astype(np.float32)
    return [s * jnp.sin((n + 1) * np.float32(np.pi) * r_ / cutoff) / r_
            for n in range(N_MAX + 1)]


def _legendre_list(c):
    polys = [jnp.ones_like(c), c]
    for l in range(2, L_MAX + 1):
        polys.append(((2 * l - 1) * c * polys[-1] - (l - 1) * polys[-2]) / l)
    return polys


def _k_bas_edge(dist3d):
    def body(r_ref, *outs):
        r = r_ref[...]
        e0 = _bessel_list(r, CUTOFF)
        fc = _poly_cutoff(r, CUTOFF)
        for n in range(N_MAX + 1):
            outs[n][...] = e0[n]
            outs[5 + n][...] = e0[n] * fc

    return pl.pallas_call(
        body,
        grid=(GE,),
        in_specs=[pl.BlockSpec((1, 2, 128), lambda i: (i, 0, 0))],
        out_specs=[pl.BlockSpec((1, 2, 128), lambda i: (i, 0, 0))] * 10,
        out_shape=[jax.ShapeDtypeStruct((GE, 2, 128), jnp.float32)] * 10,
    )(dist3d)


def _k_bas_ang(norm3d, cos3d):
    def body(r_ref, c_ref, *outs):
        r = r_ref[...]
        c = c_ref[...]
        rad = _bessel_list(r, CUT3)
        leg = _legendre_list(c)
        fc3 = _poly_cutoff(r, CUT3)
        for n in range(N_MAX + 1):
            outs[n][...] = rad[n] * fc3
            outs[5 + n][...] = leg[n]

    return pl.pallas_call(
        body,
        grid=(GA,),
        in_specs=[pl.BlockSpec((1, 2, 128), lambda i: (i, 0, 0))] * 2,
        out_specs=[pl.BlockSpec((1, 2, 128), lambda i: (i, 0, 0))] * 10,
        out_shape=[jax.ShapeDtypeStruct((GA, 2, 128), jnp.float32)] * 10,
    )(norm3d, cos3d)


def _k_enc(E0s, enc_Wp, enc_b, We3_0):
    def body(e0_ref, w_ref, b_ref, w3_ref, e_ref, t_ref):
        acc = jnp.dot(e0_ref[...], w_ref[...], precision=_PREC) + b_ref[...]
        e = _swish(acc)
        e_ref[...] = e
        t_ref[...] = _swish(jnp.dot(e, w3_ref[...], precision=_PREC))

    return pl.pallas_call(
        body,
        grid=(GE,),
        in_specs=[pl.BlockSpec((R, 16), lambda i: (i, 0)),
                  pl.BlockSpec((16, F), lambda i: (0, 0)),
                  pl.BlockSpec((1, F), lambda i: (0, 0)),
                  pl.BlockSpec((F, F), lambda i: (0, 0))],
        out_specs=[pl.BlockSpec((R, F), lambda i: (i, 0)),
                   pl.BlockSpec((R, F), lambda i: (i, 0))],
        out_shape=[jax.ShapeDtypeStruct((EP, F), jnp.float32),
                   jax.ShapeDtypeStruct((EP, F), jnp.float32)],
    )(E0s, enc_Wp, enc_b, We3_0)


def _k_emb(atomic_col, emb_pad):
    def body(a_ref, w_ref, o_ref):
        a = a_ref[...]
        lanes = lax.broadcasted_iota(jnp.int32, (1, F), 1)
        oh = (a == lanes).astype(jnp.float32)
        o_ref[...] = jnp.dot(oh, w_ref[...], precision=_PREC)

    return pl.pallas_call(
        body,
        grid=(GN,),
        in_specs=[pl.BlockSpec((R, 1), lambda i: (i, 0)),
                  pl.BlockSpec((F, F), lambda i: (0, 0))],
        out_specs=pl.BlockSpec((R, F), lambda i: (i, 0)),
        out_shape=jax.ShapeDtypeStruct((NP, F), jnp.float32),
    )(atomic_col, emb_pad)


def _k_msg3_cumsum(P, g, SA, SB, Wang_pad, Ltri):
    def body(p_ref, g_ref, sa_ref, sb_ref, w_ref, l_ref, c_ref, carry):
        i = pl.program_id(0)

        @pl.when(i == 0)
        def _():
            carry[...] = jnp.zeros((8, F), jnp.float32)

        p = p_ref[...]
        ang = (jnp.dot(p, sa_ref[...], precision=_PREC)
               * jnp.dot(p, sb_ref[...], precision=_PREC))
        a = jnp.dot(ang, w_ref[...], precision=_PREC)
        msg = a * g_ref[...]
        cv = carry[0:1, :]
        c_ref[...] = cv + jnp.dot(l_ref[...], msg, precision=_PREC)
        carry[0:1, :] = cv + jnp.sum(msg, axis=0, keepdims=True)

    return pl.pallas_call(
        body,
        grid=(GA,),
        in_specs=[pl.BlockSpec((R, 16), lambda i: (i, 0)),
                  pl.BlockSpec((R, F), lambda i: (i, 0)),
                  pl.BlockSpec((16, 32), lambda i: (0, 0)),
                  pl.BlockSpec((16, 32), lambda i: (0, 0)),
                  pl.BlockSpec((32, F), lambda i: (0, 0)),
                  pl.BlockSpec((R, R), lambda i: (0, 0))],
        out_specs=pl.BlockSpec((R, F), lambda i: (i, 0)),
        out_shape=jax.ShapeDtypeStruct((AP, F), jnp.float32),
        scratch_shapes=[pltpu.VMEM((8, F), jnp.float32)],
    )(P, g, SA, SB, Wang_pad, Ltri)


def _k_edge_node(Ga, Gb, e, xs, xd, E0s, W3o, Wedge, Wnode, WegP, WngP,
                 We3n, emit_t):
    def body(ga_ref, gb_ref, e_ref, xs_ref, xd_ref, e0_ref, w3o_ref, we_ref,
             wn_ref, weg_ref, wng_ref, we3_ref, *outs):
        if emit_t:
            e2_ref, m_ref, t_ref = outs
        else:
            e2_ref, m_ref = outs

        agg3 = gb_ref[...] - ga_ref[...]
        e1 = e_ref[...] + _swish(jnp.dot(agg3, w3o_ref[...], precision=_PREC))

        e0 = e0_ref[...]
        gate_e = jnp.dot(e0, weg_ref[...], precision=_PREC)
        gate_n = jnp.dot(e0, wng_ref[...], precision=_PREC)

        xs = xs_ref[...]
        xd = xd_ref[...]
        we = we_ref[...]
        wn = wn_ref[...]
        arg_e = (jnp.dot(xs, we[0:F, :], precision=_PREC)
                 + jnp.dot(xd, we[F:2 * F, :], precision=_PREC)
                 + jnp.dot(e1, we[2 * F:3 * F, :], precision=_PREC))
        e2 = e1 + _swish(arg_e) * gate_e
        arg_n = (jnp.dot(xs, wn[0:F, :], precision=_PREC)
                 + jnp.dot(xd, wn[F:2 * F, :], precision=_PREC)
                 + jnp.dot(e1, wn[2 * F:3 * F, :], precision=_PREC))
        msg = _swish(arg_n) * gate_n

        m_ref[...] = msg
        e2_ref[...] = e2
        if emit_t:
            t_ref[...] = _swish(jnp.dot(e2, we3_ref[...], precision=_PREC))

    n_out = 3 if emit_t else 2
    return pl.pallas_call(
        body,
        grid=(GE,),
        in_specs=[pl.BlockSpec((R, F), lambda i: (i, 0)),
                  pl.BlockSpec((R, F), lambda i: (i, 0)),
                  pl.BlockSpec((R, F), lambda i: (i, 0)),
                  pl.BlockSpec((R, F), lambda i: (i, 0)),
                  pl.BlockSpec((R, F), lambda i: (i, 0)),
                  pl.BlockSpec((R, 16), lambda i: (i, 0)),
                  pl.BlockSpec((F, F), lambda i: (0, 0)),
                  pl.BlockSpec((3 * F, F), lambda i: (0, 0)),
                  pl.BlockSpec((3 * F, F), lambda i: (0, 0)),
                  pl.BlockSpec((16, F), lambda i: (0, 0)),
                  pl.BlockSpec((16, F), lambda i: (0, 0)),
                  pl.BlockSpec((F, F), lambda i: (0, 0))],
        out_specs=[pl.BlockSpec((R, F), lambda i: (i, 0))] * n_out,
        out_shape=[jax.ShapeDtypeStruct((EP, F), jnp.float32)] * n_out,
    )(Ga, Gb, e, xs, xd, E0s, W3o, Wedge, Wnode, WegP, WngP, We3n)


def _k_xupd(x, partials):
    def body(x_ref, a_ref, b_ref, o_ref):
        o_ref[...] = x_ref[...] + a_ref[0] + b_ref[0]

    return pl.pallas_call(
        body,
        grid=(GN,),
        in_specs=[pl.BlockSpec((R, F), lambda i: (i, 0)),
                  pl.BlockSpec((1, R, F), lambda i: (0, i, 0)),
                  pl.BlockSpec((1, R, F), lambda i: (1, i, 0))],
        out_specs=pl.BlockSpec((R, F), lambda i: (i, 0)),
        out_shape=jax.ShapeDtypeStruct((NP, F), jnp.float32),
    )(x, partials, partials)


def _k_out(x, eW1, eb1, eW2, eb2, eW3_row):
    def body(x_ref, w1_ref, b1_ref, w2_ref, b2_ref, w3_ref, o_ref):
        h = _swish(jnp.dot(x_ref[...], w1_ref[...], precision=_PREC)
                   + b1_ref[...])
        h = _swish(jnp.dot(h, w2_ref[...], precision=_PREC) + b2_ref[...])
        o_ref[...] = jnp.sum(h * w3_ref[...], axis=1, keepdims=True)

    return pl.pallas_call(
        body,
        grid=(GN,),
        in_specs=[pl.BlockSpec((R, F), lambda i: (i, 0)),
                  pl.BlockSpec((F, F), lambda i: (0, 0)),
                  pl.BlockSpec((1, F), lambda i: (0, 0)),
                  pl.BlockSpec((F, F), lambda i: (0, 0)),
                  pl.BlockSpec((1, F), lambda i: (0, 0)),
                  pl.BlockSpec((1, F), lambda i: (0, 0))],
        out_specs=pl.BlockSpec((R, 1), lambda i: (i, 0)),
        out_shape=jax.ShapeDtypeStruct((NP, 1), jnp.float32),
    )(x, eW1, eb1, eW2, eb2, eW3_row)


_NW = 32
_CH = 128


def _gather_one_call(table, idx, width=F):
    B = idx.shape[0]
    per = B // _NW
    halfA = ((per // 2) // 8) * 8
    lenB = per - halfA
    nf = min(halfA // _CH, lenB // _CH)

    def _tail_chunks(start, length):
        out = []
        done = nf * _CH
        while done < length:
            n = min(_CH, length - done)
            out.append((start + done, n))
            done += n
        return out

    mesh = plsc.VectorSubcoreMesh(core_axis_name="c", subcore_axis_name="s")

    @functools.partial(
        pl.kernel, mesh=mesh,
        out_type=jax.ShapeDtypeStruct((B, width), jnp.float32),
        scratch_types=[pltpu.VMEM((_CH,), jnp.int32),
                       pltpu.VMEM((_CH, width), jnp.float32),
                       pltpu.VMEM((_CH,), jnp.int32),
                       pltpu.VMEM((_CH, width), jnp.float32),
                       pltpu.SemaphoreType.DMA,
                       pltpu.SemaphoreType.DMA],
    )
    def k(tab, ih, oh, iva, rva, ivb, rvb, sa, sb):
        wid = lax.axis_index("s") * 2 + lax.axis_index("c")
        base = wid * per

        def do1(off, n, iv, rv, sem):
            pltpu.sync_copy(ih.at[pl.ds(off, n)], iv.at[pl.ds(0, n)])
            pltpu.async_copy(tab.at[iv.at[pl.ds(0, n)]],
                             rv.at[pl.ds(0, n)], sem).wait()
            pltpu.sync_copy(rv.at[pl.ds(0, n)], oh.at[pl.ds(off, n)])

        def do(offa, offb, n):
            pltpu.sync_copy(ih.at[pl.ds(offa, n)], iva.at[pl.ds(0, n)])
            cpa = pltpu.async_copy(tab.at[iva.at[pl.ds(0, n)]],
                                   rva.at[pl.ds(0, n)], sa)
            pltpu.sync_copy(ih.at[pl.ds(offb, n)], ivb.at[pl.ds(0, n)])
            cpb = pltpu.async_copy(tab.at[ivb.at[pl.ds(0, n)]],
                                   rvb.at[pl.ds(0, n)], sb)
            cpa.wait()
            pltpu.sync_copy(rva.at[pl.ds(0, n)], oh.at[pl.ds(offa, n)])
            cpb.wait()
            pltpu.sync_copy(rvb.at[pl.ds(0, n)], oh.at[pl.ds(offb, n)])

        def body(i, _):
            do(base + i * _CH, base + halfA + i * _CH, _CH)
            return ()

        lax.fori_loop(0, nf, body, ())
        for off, n in _tail_chunks(base, halfA):
            do1(off, n, iva, rva, sa)
        for off, n in _tail_chunks(base + halfA, lenB):
            do1(off, n, ivb, rvb, sb)

    return k(table, idx)


def _gather_pair_call(table, idx_a, idx_b):
    B = idx_a.shape[0]
    per = B // _NW
    nfull = per // _CH
    rem = per - nfull * _CH
    mesh = plsc.VectorSubcoreMesh(core_axis_name="c", subcore_axis_name="s")

    @functools.partial(
        pl.kernel, mesh=mesh,
        out_type=(jax.ShapeDtypeStruct((B, F), jnp.float32),
                  jax.ShapeDtypeStruct((B, F), jnp.float32)),
        scratch_types=[pltpu.VMEM((_CH,), jnp.int32),
                       pltpu.VMEM((_CH, F), jnp.float32),
                       pltpu.VMEM((_CH,), jnp.int32),
                       pltpu.VMEM((_CH, F), jnp.float32),
                       pltpu.SemaphoreType.DMA,
                       pltpu.SemaphoreType.DMA],
    )
    def k(tab, ia, ib, oa, ob, iva, rva, ivb, rvb, sa, sb):
        wid = lax.axis_index("s") * 2 + lax.axis_index("c")
        base = wid * per

        def do(off, n):
            pltpu.sync_copy(ia.at[pl.ds(off, n)], iva.at[pl.ds(0, n)])
            cpa = pltpu.async_copy(tab.at[iva.at[pl.ds(0, n)]],
                                   rva.at[pl.ds(0, n)], sa)
            pltpu.sync_copy(ib.at[pl.ds(off, n)], ivb.at[pl.ds(0, n)])
            cpb = pltpu.async_copy(tab.at[ivb.at[pl.ds(0, n)]],
                                   rvb.at[pl.ds(0, n)], sb)
            cpa.wait()
            pltpu.sync_copy(rva.at[pl.ds(0, n)], oa.at[pl.ds(off, n)])
            cpb.wait()
            pltpu.sync_copy(rvb.at[pl.ds(0, n)], ob.at[pl.ds(off, n)])

        def body(i, _):
            do(base + i * _CH, _CH)
            return ()

        lax.fori_loop(0, nfull, body, ())
        if rem:
            do(base + nfull * _CH, rem)

    return k(table, idx_a, idx_b)


def _scatter_add_call(msg, dst_idx, zeros_hbm):
    per_core = EP // 2
    per_sub = per_core // 16
    nf = per_sub // _CH
    rem = per_sub - nf * _CH
    rows_sub = NP // 16
    mesh = plsc.VectorSubcoreMesh(core_axis_name="c", subcore_axis_name="s")

    @functools.partial(
        pl.kernel, mesh=mesh,
        out_type=jax.ShapeDtypeStruct((2, NP, F), jnp.float32),
        scratch_types=[pltpu.VMEM((_CH,), jnp.int32),
                       pltpu.VMEM((_CH, F), jnp.float32),
                       pltpu.VMEM_SHARED((NP, F), jnp.float32)],
    )
    def k(msg_h, idx_h, zero_h, out_h, iv, rv, shared):
        c = lax.axis_index("c")
        sid = lax.axis_index("s")
        pltpu.sync_copy(zero_h.at[pl.ds(sid * rows_sub, rows_sub)],
                        shared.at[pl.ds(sid * rows_sub, rows_sub)])
        plsc.subcore_barrier()
        base = c * per_core + sid * per_sub

        def do(off, n):
            pltpu.sync_copy(idx_h.at[pl.ds(off, n)], iv.at[pl.ds(0, n)])
            pltpu.sync_copy(msg_h.at[pl.ds(off, n)], rv.at[pl.ds(0, n)])
            pltpu.sync_copy(rv.at[pl.ds(0, n)],
                            shared.at[iv.at[pl.ds(0, n)]], add=True)

        def body(i, _):
            do(base + i * _CH, _CH)
            return ()

        lax.fori_loop(0, nf, body, ())
        if rem:
            do(base + nf * _CH, rem)
        plsc.subcore_barrier()
        pltpu.sync_copy(shared.at[pl.ds(sid * rows_sub, rows_sub)],
                        out_h.at[c].at[pl.ds(sid * rows_sub, rows_sub)])

    return k(msg, dst_idx, zeros_hbm)


def _pad1(a, n, val):
    return jnp.concatenate(
        [a, jnp.full((n - a.shape[0],), val, a.dtype)])


def kernel(atomic_numbers, edge_index, edge_dist, three_body_indices, norm_ik,
           three_body_cos_angles, total_num_bonds, total_num_angles, params):
    p = params
    f32 = jnp.float32
    i32 = jnp.int32
    tbi0 = three_body_indices[:, 0].astype(i32)
    tbi1 = three_body_indices[:, 1].astype(i32)
    src = edge_index[0].astype(i32)
    dst = edge_index[1].astype(i32)

    _, norm_s, cos_s, tbi1_s = lax.sort(
        (tbi0, norm_ik.astype(f32), three_body_cos_angles.astype(f32), tbi1),
        num_keys=1)
    cnt_a = jnp.zeros((N_EDGES,), i32).at[tbi0].add(1)
    csa = jnp.cumsum(cnt_a)
    rsA_a = _pad1(jnp.concatenate([jnp.zeros((1,), i32), csa[:-1]]),
                  EP, N_ANGLES)
    rsB_a = _pad1(csa, EP, N_ANGLES)

    dist3d = _pad1(edge_dist.astype(f32), EP, 10.0).reshape(GE, 2, 128)
    norm3d = _pad1(norm_s, AP, 10.0).reshape(GA, 2, 128)
    cos3d = _pad1(cos_s, AP, 0.0).reshape(GA, 2, 128)

    eb = _k_bas_edge(dist3d)
    ab = _k_bas_ang(norm3d, cos3d)
    E0s = jnp.stack([o.reshape(EP) for o in eb], axis=1)
    E0s = jnp.concatenate([E0s, jnp.zeros((EP, 6), f32)], axis=1)
    Ps = jnp.stack([o.reshape(AP) for o in ab], axis=1)
    Ps = jnp.concatenate([Ps, jnp.zeros((AP, 6), f32)], axis=1)

    src_p = _pad1(src, EP, 0)
    dst_p = _pad1(dst, EP, 0)
    tbi1_p = _pad1(tbi1_s, AP, 0)

    emb_pad = jnp.zeros((F, F), f32).at[:NUM_EL].set(p["emb"].astype(f32))
    enc_Wp = jnp.zeros((16, F), f32).at[:N_MAX + 1].set(p["enc_W"].astype(f32))
    enc_b = p["enc_b"].astype(f32)[None, :]
    Ltri = jnp.asarray(np.tril(np.ones((R, R), np.float32), -1))
    sa_np = np.zeros((16, 32), np.float32)
    sb_np = np.zeros((16, 32), np.float32)
    for l in range(L_MAX + 1):
        for n in range(N_MAX + 1):
            sa_np[n, l * 5 + n] = 1.0
            sb_np[5 + l, l * 5 + n] = 1.0
    SA = jnp.asarray(sa_np)
    SB = jnp.asarray(sb_np)
    zeros_np = jnp.zeros((NP, F), f32)

    blocks = p["blocks"]
    Wang_pads = [jnp.zeros((32, F), f32).at[:25].set(b["Wang"].astype(f32))
                 for b in blocks]
    WegPs = [jnp.zeros((16, F), f32).at[5:10].set(b["Weg"].astype(f32))
             for b in blocks]
    WngPs = [jnp.zeros((16, F), f32).at[5:10].set(b["Wng"].astype(f32))
             for b in blocks]

    atomic_col = _pad1(atomic_numbers.astype(i32), NP, 0)[:, None]
    x = _k_emb(atomic_col, emb_pad)
    e, t = _k_enc(E0s, enc_Wp, enc_b, blocks[0]["We3"].astype(f32))

    for b in range(NBLOCKS):
        blk = blocks[b]
        g = _gather_one_call(t, tbi1_p)
        C = _k_msg3_cumsum(Ps, g, SA, SB, Wang_pads[b], Ltri)
        Ga, Gb = _gather_pair_call(C, rsA_a, rsB_a)
        xs, xd = _gather_pair_call(x, src_p, dst_p)
        emit_t = b < NBLOCKS - 1
        We3n = (blocks[b + 1]["We3"] if emit_t else blocks[0]["We3"]).astype(f32)
        outs = _k_edge_node(Ga, Gb, e, xs, xd, E0s, blk["W3o"].astype(f32),
                            blk["Wedge"].astype(f32), blk["Wnode"].astype(f32),
                            WegPs[b], WngPs[b], We3n, emit_t)
        if emit_t:
            e, msg, t = outs
        else:
            e, msg = outs
        partials = _scatter_add_call(msg, dst_p, zeros_np)
        x = _k_xupd(x, partials)

    energy = _k_out(x, p["eW1"].astype(f32), p["eb1"].astype(f32)[None, :],
                    p["eW2"].astype(f32), p["eb2"].astype(f32)[None, :],
                    p["eW3"].astype(f32)[:, 0][None, :])
    return energy[:N_NODES] + p["eb3"].astype(f32)[None, :]


# speedup vs baseline: 2.7210x; 1.0445x over previous
import functools

import jax
import jax.numpy as jnp
import numpy as np
from jax import lax
from jax.experimental import pallas as pl
from jax.experimental.pallas import tpu as pltpu
from jax.experimental.pallas import tpu_sc as plsc

N_NODES = 10000
N_EDGES = 160000
N_ANGLES = 400000
F = 128
L_MAX = 4
N_MAX = 4
CUTOFF = 5.0
CUT3 = 4.0
NUM_EL = 108
NBLOCKS = 4

R = 256
EP = 160512
AP = 400384
NP = 10240
GE = EP // R
GA = AP // R
GN = NP // R

_PREC = jax.lax.Precision.DEFAULT


def _swish(x):
    return x * (0.5 * jnp.tanh(0.5 * x) + 0.5)


def _poly_cutoff(r, c):
    t = jnp.clip(r / c, 0.0, 1.0)
    return 1.0 - 6.0 * t ** 5 + 15.0 * t ** 4 - 10.0 * t ** 3


def _bessel_list(r, cutoff):
    r_ = r + 1e-8
    s = np.sqrt(2.0 / cutoff).astype(np.float32)
    return [s * jnp.sin((n + 1) * np.float32(np.pi) * r_ / cutoff) / r_
            for n in range(N_MAX + 1)]


def _legendre_list(c):
    polys = [jnp.ones_like(c), c]
    for l in range(2, L_MAX + 1):
        polys.append(((2 * l - 1) * c * polys[-1] - (l - 1) * polys[-2]) / l)
    return polys


def _k_bas_edge(dist3d):
    def body(r_ref, *outs):
        r = r_ref[...]
        e0 = _bessel_list(r, CUTOFF)
        fc = _poly_cutoff(r, CUTOFF)
        for n in range(N_MAX + 1):
            outs[n][...] = e0[n]
            outs[5 + n][...] = e0[n] * fc

    return pl.pallas_call(
        body,
        grid=(GE // 3,),
        in_specs=[pl.BlockSpec((3, 2, 128), lambda i: (i, 0, 0))],
        out_specs=[pl.BlockSpec((3, 2, 128), lambda i: (i, 0, 0))] * 10,
        out_shape=[jax.ShapeDtypeStruct((GE, 2, 128), jnp.float32)] * 10,
    )(dist3d)


def _k_bas_ang(norm3d, cos3d):
    def body(r_ref, c_ref, *outs):
        r = r_ref[...]
        c = c_ref[...]
        rad = _bessel_list(r, CUT3)
        leg = _legendre_list(c)
        fc3 = _poly_cutoff(r, CUT3)
        for n in range(N_MAX + 1):
            outs[n][...] = rad[n] * fc3
            outs[5 + n][...] = leg[n]

    return pl.pallas_call(
        body,
        grid=(GA // 4,),
        in_specs=[pl.BlockSpec((4, 2, 128), lambda i: (i, 0, 0))] * 2,
        out_specs=[pl.BlockSpec((4, 2, 128), lambda i: (i, 0, 0))] * 10,
        out_shape=[jax.ShapeDtypeStruct((GA, 2, 128), jnp.float32)] * 10,
    )(norm3d, cos3d)


def _k_enc(E0s, enc_Wp, enc_b, We3_0):
    def body(e0_ref, w_ref, b_ref, w3_ref, e_ref, t_ref):
        acc = jnp.dot(e0_ref[...], w_ref[...], precision=_PREC) + b_ref[...]
        e = _swish(acc)
        e_ref[...] = e
        t_ref[...] = _swish(jnp.dot(e, w3_ref[...], precision=_PREC))

    return pl.pallas_call(
        body,
        grid=(GE,),
        in_specs=[pl.BlockSpec((R, 16), lambda i: (i, 0)),
                  pl.BlockSpec((16, F), lambda i: (0, 0)),
                  pl.BlockSpec((1, F), lambda i: (0, 0)),
                  pl.BlockSpec((F, F), lambda i: (0, 0))],
        out_specs=[pl.BlockSpec((R, F), lambda i: (i, 0)),
                   pl.BlockSpec((R, F), lambda i: (i, 0))],
        out_shape=[jax.ShapeDtypeStruct((EP, F), jnp.float32),
                   jax.ShapeDtypeStruct((EP, F), jnp.float32)],
    )(E0s, enc_Wp, enc_b, We3_0)


def _k_emb(atomic_col, emb_pad):
    def body(a_ref, w_ref, o_ref):
        a = a_ref[...]
        lanes = lax.broadcasted_iota(jnp.int32, (1, F), 1)
        oh = (a == lanes).astype(jnp.float32)
        o_ref[...] = jnp.dot(oh, w_ref[...], precision=_PREC)

    return pl.pallas_call(
        body,
        grid=(GN,),
        in_specs=[pl.BlockSpec((R, 1), lambda i: (i, 0)),
                  pl.BlockSpec((F, F), lambda i: (0, 0))],
        out_specs=pl.BlockSpec((R, F), lambda i: (i, 0)),
        out_shape=jax.ShapeDtypeStruct((NP, F), jnp.float32),
    )(atomic_col, emb_pad)


def _k_msg3_cumsum(P, g, SA, SB, Wang_pad, Ltri):
    def body(p_ref, g_ref, sa_ref, sb_ref, w_ref, l_ref, c_ref, carry):
        i = pl.program_id(0)

        @pl.when(i == 0)
        def _():
            carry[...] = jnp.zeros((8, F), jnp.float32)

        p = p_ref[...]
        ang = (jnp.dot(p, sa_ref[...], precision=_PREC)
               * jnp.dot(p, sb_ref[...], precision=_PREC))
        a = jnp.dot(ang, w_ref[...], precision=_PREC)
        msg = a * g_ref[...]
        cv = carry[0:1, :]
        c_ref[...] = cv + jnp.dot(l_ref[...], msg.astype(jnp.bfloat16),
                                  preferred_element_type=jnp.float32)
        carry[0:1, :] = cv + jnp.sum(msg, axis=0, keepdims=True)

    return pl.pallas_call(
        body,
        grid=(GA,),
        in_specs=[pl.BlockSpec((R, 16), lambda i: (i, 0)),
                  pl.BlockSpec((R, F), lambda i: (i, 0)),
                  pl.BlockSpec((16, 32), lambda i: (0, 0)),
                  pl.BlockSpec((16, 32), lambda i: (0, 0)),
                  pl.BlockSpec((32, F), lambda i: (0, 0)),
                  pl.BlockSpec((R, R), lambda i: (0, 0))],
        out_specs=pl.BlockSpec((R, F), lambda i: (i, 0)),
        out_shape=jax.ShapeDtypeStruct((AP, F), jnp.float32),
        scratch_shapes=[pltpu.VMEM((8, F), jnp.float32)],
    )(P, g, SA, SB, Wang_pad, Ltri)


def _k_edge_node(Ga, Gb, e, xs, xd, E0s, W3o, Wedge, Wnode, WegP, WngP,
                 We3n, emit_t):
    def body(ga_ref, gb_ref, e_ref, xs_ref, xd_ref, e0_ref, w3o_ref, we_ref,
             wn_ref, weg_ref, wng_ref, we3_ref, *outs):
        if emit_t:
            e2_ref, m_ref, t_ref = outs
        else:
            e2_ref, m_ref = outs

        agg3 = gb_ref[...] - ga_ref[...]
        e1 = e_ref[...] + _swish(jnp.dot(agg3, w3o_ref[...], precision=_PREC))

        e0 = e0_ref[...]
        gate_e = jnp.dot(e0, weg_ref[...], precision=_PREC)
        gate_n = jnp.dot(e0, wng_ref[...], precision=_PREC)

        xs = xs_ref[...]
        xd = xd_ref[...]
        we = we_ref[...]
        wn = wn_ref[...]
        arg_e = (jnp.dot(xs, we[0:F, :], precision=_PREC)
                 + jnp.dot(xd, we[F:2 * F, :], precision=_PREC)
                 + jnp.dot(e1, we[2 * F:3 * F, :], precision=_PREC))
        e2 = e1 + _swish(arg_e) * gate_e
        arg_n = (jnp.dot(xs, wn[0:F, :], precision=_PREC)
                 + jnp.dot(xd, wn[F:2 * F, :], precision=_PREC)
                 + jnp.dot(e1, wn[2 * F:3 * F, :], precision=_PREC))
        msg = _swish(arg_n) * gate_n

        m_ref[...] = msg
        e2_ref[...] = e2
        if emit_t:
            t_ref[...] = _swish(jnp.dot(e2, we3_ref[...], precision=_PREC))

    n_out = 3 if emit_t else 2
    return pl.pallas_call(
        body,
        grid=(GE,),
        in_specs=[pl.BlockSpec((R, F), lambda i: (i, 0)),
                  pl.BlockSpec((R, F), lambda i: (i, 0)),
                  pl.BlockSpec((R, F), lambda i: (i, 0)),
                  pl.BlockSpec((R, F), lambda i: (i, 0)),
                  pl.BlockSpec((R, F), lambda i: (i, 0)),
                  pl.BlockSpec((R, 16), lambda i: (i, 0)),
                  pl.BlockSpec((F, F), lambda i: (0, 0)),
                  pl.BlockSpec((3 * F, F), lambda i: (0, 0)),
                  pl.BlockSpec((3 * F, F), lambda i: (0, 0)),
                  pl.BlockSpec((16, F), lambda i: (0, 0)),
                  pl.BlockSpec((16, F), lambda i: (0, 0)),
                  pl.BlockSpec((F, F), lambda i: (0, 0))],
        out_specs=[pl.BlockSpec((R, F), lambda i: (i, 0))] * n_out,
        out_shape=[jax.ShapeDtypeStruct((EP, F), jnp.float32)] * n_out,
    )(Ga, Gb, e, xs, xd, E0s, W3o, Wedge, Wnode, WegP, WngP, We3n)


def _k_xupd(x, partials):
    def body(x_ref, a_ref, b_ref, o_ref):
        o_ref[...] = x_ref[...] + a_ref[0] + b_ref[0]

    return pl.pallas_call(
        body,
        grid=(GN,),
        in_specs=[pl.BlockSpec((R, F), lambda i: (i, 0)),
                  pl.BlockSpec((1, R, F), lambda i: (0, i, 0)),
                  pl.BlockSpec((1, R, F), lambda i: (1, i, 0))],
        out_specs=pl.BlockSpec((R, F), lambda i: (i, 0)),
        out_shape=jax.ShapeDtypeStruct((NP, F), jnp.float32),
    )(x, partials, partials)


def _k_out(x, eW1, eb1, eW2, eb2, eW3_row):
    def body(x_ref, w1_ref, b1_ref, w2_ref, b2_ref, w3_ref, o_ref):
        h = _swish(jnp.dot(x_ref[...], w1_ref[...], precision=_PREC)
                   + b1_ref[...])
        h = _swish(jnp.dot(h, w2_ref[...], precision=_PREC) + b2_ref[...])
        o_ref[...] = jnp.sum(h * w3_ref[...], axis=1, keepdims=True)

    return pl.pallas_call(
        body,
        grid=(GN,),
        in_specs=[pl.BlockSpec((R, F), lambda i: (i, 0)),
                  pl.BlockSpec((F, F), lambda i: (0, 0)),
                  pl.BlockSpec((1, F), lambda i: (0, 0)),
                  pl.BlockSpec((F, F), lambda i: (0, 0)),
                  pl.BlockSpec((1, F), lambda i: (0, 0)),
                  pl.BlockSpec((1, F), lambda i: (0, 0))],
        out_specs=pl.BlockSpec((R, 1), lambda i: (i, 0)),
        out_shape=jax.ShapeDtypeStruct((NP, 1), jnp.float32),
    )(x, eW1, eb1, eW2, eb2, eW3_row)


_NW = 32
_CH = 128


def _gather_one_call(table, idx, width=F):
    B = idx.shape[0]
    per = B // _NW
    halfA = ((per // 2) // 8) * 8
    lenB = per - halfA
    nf = min(halfA // _CH, lenB // _CH)

    def _tail_chunks(start, length):
        out = []
        done = nf * _CH
        while done < length:
            n = min(_CH, length - done)
            out.append((start + done, n))
            done += n
        return out

    mesh = plsc.VectorSubcoreMesh(core_axis_name="c", subcore_axis_name="s")

    @functools.partial(
        pl.kernel, mesh=mesh,
        out_type=jax.ShapeDtypeStruct((B, width), jnp.float32),
        scratch_types=[pltpu.VMEM((_CH,), jnp.int32),
                       pltpu.VMEM((_CH, width), jnp.float32),
                       pltpu.VMEM((_CH,), jnp.int32),
                       pltpu.VMEM((_CH, width), jnp.float32),
                       pltpu.SemaphoreType.DMA,
                       pltpu.SemaphoreType.DMA],
    )
    def k(tab, ih, oh, iva, rva, ivb, rvb, sa, sb):
        wid = lax.axis_index("s") * 2 + lax.axis_index("c")
        base = wid * per

        def do1(off, n, iv, rv, sem):
            pltpu.sync_copy(ih.at[pl.ds(off, n)], iv.at[pl.ds(0, n)])
            pltpu.async_copy(tab.at[iv.at[pl.ds(0, n)]],
                             rv.at[pl.ds(0, n)], sem).wait()
            pltpu.sync_copy(rv.at[pl.ds(0, n)], oh.at[pl.ds(off, n)])

        def do(offa, offb, n):
            pltpu.sync_copy(ih.at[pl.ds(offa, n)], iva.at[pl.ds(0, n)])
            cpa = pltpu.async_copy(tab.at[iva.at[pl.ds(0, n)]],
                                   rva.at[pl.ds(0, n)], sa)
            pltpu.sync_copy(ih.at[pl.ds(offb, n)], ivb.at[pl.ds(0, n)])
            cpb = pltpu.async_copy(tab.at[ivb.at[pl.ds(0, n)]],
                                   rvb.at[pl.ds(0, n)], sb)
            cpa.wait()
            pltpu.sync_copy(rva.at[pl.ds(0, n)], oh.at[pl.ds(offa, n)])
            cpb.wait()
            pltpu.sync_copy(rvb.at[pl.ds(0, n)], oh.at[pl.ds(offb, n)])

        def body(i, _):
            do(base + i * _CH, base + halfA + i * _CH, _CH)
            return ()

        lax.fori_loop(0, nf, body, ())
        for off, n in _tail_chunks(base, halfA):
            do1(off, n, iva, rva, sa)
        for off, n in _tail_chunks(base + halfA, lenB):
            do1(off, n, ivb, rvb, sb)

    return k(table, idx)


def _gather_pair_call(table, idx_a, idx_b):
    B = idx_a.shape[0]
    per = B // _NW
    nfull = per // _CH
    rem = per - nfull * _CH
    mesh = plsc.VectorSubcoreMesh(core_axis_name="c", subcore_axis_name="s")

    @functools.partial(
        pl.kernel, mesh=mesh,
        out_type=(jax.ShapeDtypeStruct((B, F), jnp.float32),
                  jax.ShapeDtypeStruct((B, F), jnp.float32)),
        scratch_types=[pltpu.VMEM((_CH,), jnp.int32),
                       pltpu.VMEM((_CH, F), jnp.float32),
                       pltpu.VMEM((_CH,), jnp.int32),
                       pltpu.VMEM((_CH, F), jnp.float32),
                       pltpu.SemaphoreType.DMA,
                       pltpu.SemaphoreType.DMA],
    )
    def k(tab, ia, ib, oa, ob, iva, rva, ivb, rvb, sa, sb):
        wid = lax.axis_index("s") * 2 + lax.axis_index("c")
        base = wid * per

        def do(off, n):
            pltpu.sync_copy(ia.at[pl.ds(off, n)], iva.at[pl.ds(0, n)])
            cpa = pltpu.async_copy(tab.at[iva.at[pl.ds(0, n)]],
                                   rva.at[pl.ds(0, n)], sa)
            pltpu.sync_copy(ib.at[pl.ds(off, n)], ivb.at[pl.ds(0, n)])
            cpb = pltpu.async_copy(tab.at[ivb.at[pl.ds(0, n)]],
                                   rvb.at[pl.ds(0, n)], sb)
            cpa.wait()
            pltpu.sync_copy(rva.at[pl.ds(0, n)], oa.at[pl.ds(off, n)])
            cpb.wait()
            pltpu.sync_copy(rvb.at[pl.ds(0, n)], ob.at[pl.ds(off, n)])

        def body(i, _):
            do(base + i * _CH, _CH)
            return ()

        lax.fori_loop(0, nfull, body, ())
        if rem:
            do(base + nfull * _CH, rem)

    return k(table, idx_a, idx_b)


def _scatter_add_call(msg, dst_idx, zeros_hbm):
    per_core = EP // 2
    per_sub = per_core // 16
    nf = per_sub // _CH
    rem = per_sub - nf * _CH
    rows_sub = NP // 16
    mesh = plsc.VectorSubcoreMesh(core_axis_name="c", subcore_axis_name="s")

    @functools.partial(
        pl.kernel, mesh=mesh,
        out_type=jax.ShapeDtypeStruct((2, NP, F), jnp.float32),
        scratch_types=[pltpu.VMEM((_CH,), jnp.int32),
                       pltpu.VMEM((_CH, F), jnp.float32),
                       pltpu.VMEM_SHARED((NP, F), jnp.float32)],
    )
    def k(msg_h, idx_h, zero_h, out_h, iv, rv, shared):
        c = lax.axis_index("c")
        sid = lax.axis_index("s")
        pltpu.sync_copy(zero_h.at[pl.ds(sid * rows_sub, rows_sub)],
                        shared.at[pl.ds(sid * rows_sub, rows_sub)])
        plsc.subcore_barrier()
        base = c * per_core + sid * per_sub

        def do(off, n):
            pltpu.sync_copy(idx_h.at[pl.ds(off, n)], iv.at[pl.ds(0, n)])
            pltpu.sync_copy(msg_h.at[pl.ds(off, n)], rv.at[pl.ds(0, n)])
            pltpu.sync_copy(rv.at[pl.ds(0, n)],
                            shared.at[iv.at[pl.ds(0, n)]], add=True)

        def body(i, _):
            do(base + i * _CH, _CH)
            return ()

        lax.fori_loop(0, nf, body, ())
        if rem:
            do(base + nf * _CH, rem)
        plsc.subcore_barrier()
        pltpu.sync_copy(shared.at[pl.ds(sid * rows_sub, rows_sub)],
                        out_h.at[c].at[pl.ds(sid * rows_sub, rows_sub)])

    return k(msg, dst_idx, zeros_hbm)


def _pad1(a, n, val):
    return jnp.concatenate(
        [a, jnp.full((n - a.shape[0],), val, a.dtype)])


def kernel(atomic_numbers, edge_index, edge_dist, three_body_indices, norm_ik,
           three_body_cos_angles, total_num_bonds, total_num_angles, params):
    p = params
    f32 = jnp.float32
    i32 = jnp.int32
    tbi0 = three_body_indices[:, 0].astype(i32)
    tbi1 = three_body_indices[:, 1].astype(i32)
    src = edge_index[0].astype(i32)
    dst = edge_index[1].astype(i32)

    _, norm_s, cos_s, tbi1_s = lax.sort(
        (tbi0, norm_ik.astype(f32), three_body_cos_angles.astype(f32), tbi1),
        num_keys=1)
    cnt_a = jnp.zeros((N_EDGES,), i32).at[tbi0].add(1)
    csa = jnp.cumsum(cnt_a)
    rsA_a = _pad1(jnp.concatenate([jnp.zeros((1,), i32), csa[:-1]]),
                  EP, N_ANGLES)
    rsB_a = _pad1(csa, EP, N_ANGLES)

    dist3d = _pad1(edge_dist.astype(f32), EP, 10.0).reshape(GE, 2, 128)
    norm3d = _pad1(norm_s, AP, 10.0).reshape(GA, 2, 128)
    cos3d = _pad1(cos_s, AP, 0.0).reshape(GA, 2, 128)

    eb = _k_bas_edge(dist3d)
    ab = _k_bas_ang(norm3d, cos3d)
    E0s = jnp.stack([o.reshape(EP) for o in eb], axis=1)
    E0s = jnp.concatenate([E0s, jnp.zeros((EP, 6), f32)], axis=1)
    Ps = jnp.stack([o.reshape(AP) for o in ab], axis=1)
    Ps = jnp.concatenate([Ps, jnp.zeros((AP, 6), f32)], axis=1)

    src_p = _pad1(src, EP, 0)
    dst_p = _pad1(dst, EP, 0)
    tbi1_p = _pad1(tbi1_s, AP, 0)

    emb_pad = jnp.zeros((F, F), f32).at[:NUM_EL].set(p["emb"].astype(f32))
    enc_Wp = jnp.zeros((16, F), f32).at[:N_MAX + 1].set(p["enc_W"].astype(f32))
    enc_b = p["enc_b"].astype(f32)[None, :]
    Ltri = jnp.asarray(np.tril(np.ones((R, R), np.float32), -1)).astype(jnp.bfloat16)
    sa_np = np.zeros((16, 32), np.float32)
    sb_np = np.zeros((16, 32), np.float32)
    for l in range(L_MAX + 1):
        for n in range(N_MAX + 1):
            sa_np[n, l * 5 + n] = 1.0
            sb_np[5 + l, l * 5 + n] = 1.0
    SA = jnp.asarray(sa_np)
    SB = jnp.asarray(sb_np)
    zeros_np = jnp.zeros((NP, F), f32)

    blocks = p["blocks"]
    Wang_pads = [jnp.zeros((32, F), f32).at[:25].set(b["Wang"].astype(f32))
                 for b in blocks]
    WegPs = [jnp.zeros((16, F), f32).at[5:10].set(b["Weg"].astype(f32))
             for b in blocks]
    WngPs = [jnp.zeros((16, F), f32).at[5:10].set(b["Wng"].astype(f32))
             for b in blocks]

    atomic_col = _pad1(atomic_numbers.astype(i32), NP, 0)[:, None]
    x = _k_emb(atomic_col, emb_pad)
    e, t = _k_enc(E0s, enc_Wp, enc_b, blocks[0]["We3"].astype(f32))

    for b in range(NBLOCKS):
        blk = blocks[b]
        g = _gather_one_call(t, tbi1_p)
        C = _k_msg3_cumsum(Ps, g, SA, SB, Wang_pads[b], Ltri)
        Ga, Gb = _gather_pair_call(C, rsA_a, rsB_a)
        xs, xd = _gather_pair_call(x, src_p, dst_p)
        emit_t = b < NBLOCKS - 1
        We3n = (blocks[b + 1]["We3"] if emit_t else blocks[0]["We3"]).astype(f32)
        outs = _k_edge_node(Ga, Gb, e, xs, xd, E0s, blk["W3o"].astype(f32),
                            blk["Wedge"].astype(f32), blk["Wnode"].astype(f32),
                            WegPs[b], WngPs[b], We3n, emit_t)
        if emit_t:
            e, msg, t = outs
        else:
            e, msg = outs
        partials = _scatter_add_call(msg, dst_p, zeros_np)
        x = _k_xupd(x, partials)

    energy = _k_out(x, p["eW1"].astype(f32), p["eb1"].astype(f32)[None, :],
                    p["eW2"].astype(f32), p["eb2"].astype(f32)[None, :],
                    p["eW3"].astype(f32)[:, 0][None, :])
    return energy[:N_NODES] + p["eb3"].astype(f32)[None, :]


# speedup vs baseline: 3.2915x; 1.2097x over previous
import functools

import jax
import jax.numpy as jnp
import numpy as np
from jax import lax
from jax.experimental import pallas as pl
from jax.experimental.pallas import tpu as pltpu
from jax.experimental.pallas import tpu_sc as plsc

N_NODES = 10000
N_EDGES = 160000
N_ANGLES = 400000
F = 128
L_MAX = 4
N_MAX = 4
CUTOFF = 5.0
CUT3 = 4.0
NUM_EL = 108
NBLOCKS = 4

R = 256
EP = 160512
AP = 400384
NP = 10240
GE = EP // R
GA = AP // R
GN = NP // R

_PREC = jax.lax.Precision.DEFAULT


def _swish(x):
    return x * (0.5 * jnp.tanh(0.5 * x) + 0.5)


def _poly_cutoff(r, c):
    t = jnp.clip(r / c, 0.0, 1.0)
    return 1.0 - 6.0 * t ** 5 + 15.0 * t ** 4 - 10.0 * t ** 3


def _bessel_list(r, cutoff):
    r_ = r + 1e-8
    s = np.sqrt(2.0 / cutoff).astype(np.float32)
    return [s * jnp.sin((n + 1) * np.float32(np.pi) * r_ / cutoff) / r_
            for n in range(N_MAX + 1)]


def _legendre_list(c):
    polys = [jnp.ones_like(c), c]
    for l in range(2, L_MAX + 1):
        polys.append(((2 * l - 1) * c * polys[-1] - (l - 1) * polys[-2]) / l)
    return polys


def _k_bas_edge(dist3d):
    def body(r_ref, *outs):
        r = r_ref[...]
        e0 = _bessel_list(r, CUTOFF)
        fc = _poly_cutoff(r, CUTOFF)
        for n in range(N_MAX + 1):
            outs[n][...] = e0[n]
            outs[5 + n][...] = e0[n] * fc

    return pl.pallas_call(
        body,
        grid=(GE // 3,),
        in_specs=[pl.BlockSpec((3, 2, 128), lambda i: (i, 0, 0))],
        out_specs=[pl.BlockSpec((3, 2, 128), lambda i: (i, 0, 0))] * 10,
        out_shape=[jax.ShapeDtypeStruct((GE, 2, 128), jnp.float32)] * 10,
    )(dist3d)


def _k_bas_ang(norm3d, cos3d):
    def body(r_ref, c_ref, *outs):
        r = r_ref[...]
        c = c_ref[...]
        rad = _bessel_list(r, CUT3)
        leg = _legendre_list(c)
        fc3 = _poly_cutoff(r, CUT3)
        for n in range(N_MAX + 1):
            outs[n][...] = rad[n] * fc3
            outs[5 + n][...] = leg[n]

    return pl.pallas_call(
        body,
        grid=(GA // 4,),
        in_specs=[pl.BlockSpec((4, 2, 128), lambda i: (i, 0, 0))] * 2,
        out_specs=[pl.BlockSpec((4, 2, 128), lambda i: (i, 0, 0))] * 10,
        out_shape=[jax.ShapeDtypeStruct((GA, 2, 128), jnp.float32)] * 10,
    )(norm3d, cos3d)


def _k_enc(E0s, enc_Wp, enc_b, We3_0):
    def body(e0_ref, w_ref, b_ref, w3_ref, e_ref, t_ref):
        acc = jnp.dot(e0_ref[...], w_ref[...], precision=_PREC) + b_ref[...]
        e = _swish(acc)
        e_ref[...] = e
        t_ref[...] = _swish(jnp.dot(e, w3_ref[...], precision=_PREC))

    return pl.pallas_call(
        body,
        grid=(GE,),
        in_specs=[pl.BlockSpec((R, 16), lambda i: (i, 0)),
                  pl.BlockSpec((16, F), lambda i: (0, 0)),
                  pl.BlockSpec((1, F), lambda i: (0, 0)),
                  pl.BlockSpec((F, F), lambda i: (0, 0))],
        out_specs=[pl.BlockSpec((R, F), lambda i: (i, 0)),
                   pl.BlockSpec((R, F), lambda i: (i, 0))],
        out_shape=[jax.ShapeDtypeStruct((EP, F), jnp.float32),
                   jax.ShapeDtypeStruct((EP, F), jnp.float32)],
    )(E0s, enc_Wp, enc_b, We3_0)


def _k_emb(atomic_col, emb_pad):
    def body(a_ref, w_ref, o_ref):
        a = a_ref[...]
        lanes = lax.broadcasted_iota(jnp.int32, (1, F), 1)
        oh = (a == lanes).astype(jnp.float32)
        o_ref[...] = jnp.dot(oh, w_ref[...], precision=_PREC)

    return pl.pallas_call(
        body,
        grid=(GN,),
        in_specs=[pl.BlockSpec((R, 1), lambda i: (i, 0)),
                  pl.BlockSpec((F, F), lambda i: (0, 0))],
        out_specs=pl.BlockSpec((R, F), lambda i: (i, 0)),
        out_shape=jax.ShapeDtypeStruct((NP, F), jnp.float32),
    )(atomic_col, emb_pad)


def _k_msg3_cumsum(P, g, SA, SB, Wang_pad, Ltri):
    def body(p_ref, g_ref, sa_ref, sb_ref, w_ref, l_ref, c_ref, carry):
        i = pl.program_id(0)

        @pl.when(i == 0)
        def _():
            carry[...] = jnp.zeros((8, F), jnp.float32)

        p = p_ref[...]
        ang = (jnp.dot(p, sa_ref[...], precision=_PREC)
               * jnp.dot(p, sb_ref[...], precision=_PREC))
        a = jnp.dot(ang, w_ref[...], precision=_PREC)
        msg = a * g_ref[...]
        cv = carry[0:1, :]
        c_ref[...] = cv + jnp.dot(l_ref[...], msg.astype(jnp.bfloat16),
                                  preferred_element_type=jnp.float32)
        carry[0:1, :] = cv + jnp.sum(msg, axis=0, keepdims=True)

    return pl.pallas_call(
        body,
        grid=(GA // 2,),
        in_specs=[pl.BlockSpec((2 * R, 16), lambda i: (i, 0)),
                  pl.BlockSpec((2 * R, F), lambda i: (i, 0)),
                  pl.BlockSpec((16, 32), lambda i: (0, 0)),
                  pl.BlockSpec((16, 32), lambda i: (0, 0)),
                  pl.BlockSpec((32, F), lambda i: (0, 0)),
                  pl.BlockSpec((2 * R, 2 * R), lambda i: (0, 0))],
        out_specs=pl.BlockSpec((2 * R, F), lambda i: (i, 0)),
        out_shape=jax.ShapeDtypeStruct((AP, F), jnp.float32),
        scratch_shapes=[pltpu.VMEM((8, F), jnp.float32)],
    )(P, g, SA, SB, Wang_pad, Ltri)


def _k_edge_node(Ga, Gb, e, xs, xd, E0s, W3o, Wedge, Wnode, WegP, WngP,
                 We3n, emit_t):
    def body(ga_ref, gb_ref, e_ref, xs_ref, xd_ref, e0_ref, w3o_ref, we_ref,
             wn_ref, weg_ref, wng_ref, we3_ref, *outs):
        if emit_t:
            e2_ref, m_ref, t_ref = outs
        else:
            e2_ref, m_ref = outs

        agg3 = gb_ref[...] - ga_ref[...]
        e1 = e_ref[...] + _swish(jnp.dot(agg3, w3o_ref[...], precision=_PREC))

        e0 = e0_ref[...]
        gate_e = jnp.dot(e0, weg_ref[...], precision=_PREC)
        gate_n = jnp.dot(e0, wng_ref[...], precision=_PREC)

        xs = xs_ref[...]
        xd = xd_ref[...]
        we = we_ref[...]
        wn = wn_ref[...]
        arg_e = (jnp.dot(xs, we[0:F, :], precision=_PREC)
                 + jnp.dot(xd, we[F:2 * F, :], precision=_PREC)
                 + jnp.dot(e1, we[2 * F:3 * F, :], precision=_PREC))
        e2 = e1 + _swish(arg_e) * gate_e
        arg_n = (jnp.dot(xs, wn[0:F, :], precision=_PREC)
                 + jnp.dot(xd, wn[F:2 * F, :], precision=_PREC)
                 + jnp.dot(e1, wn[2 * F:3 * F, :], precision=_PREC))
        msg = _swish(arg_n) * gate_n

        m_ref[...] = msg
        e2_ref[...] = e2
        if emit_t:
            t_ref[...] = _swish(jnp.dot(e2, we3_ref[...], precision=_PREC))

    n_out = 3 if emit_t else 2
    return pl.pallas_call(
        body,
        grid=(GE,),
        in_specs=[pl.BlockSpec((R, F), lambda i: (i, 0)),
                  pl.BlockSpec((R, F), lambda i: (i, 0)),
                  pl.BlockSpec((R, F), lambda i: (i, 0)),
                  pl.BlockSpec((R, F), lambda i: (i, 0)),
                  pl.BlockSpec((R, F), lambda i: (i, 0)),
                  pl.BlockSpec((R, 16), lambda i: (i, 0)),
                  pl.BlockSpec((F, F), lambda i: (0, 0)),
                  pl.BlockSpec((3 * F, F), lambda i: (0, 0)),
                  pl.BlockSpec((3 * F, F), lambda i: (0, 0)),
                  pl.BlockSpec((16, F), lambda i: (0, 0)),
                  pl.BlockSpec((16, F), lambda i: (0, 0)),
                  pl.BlockSpec((F, F), lambda i: (0, 0))],
        out_specs=[pl.BlockSpec((R, F), lambda i: (i, 0))] * n_out,
        out_shape=[jax.ShapeDtypeStruct((EP, F), jnp.float32)] * n_out,
    )(Ga, Gb, e, xs, xd, E0s, W3o, Wedge, Wnode, WegP, WngP, We3n)


def _k_xupd(x, partials):
    def body(x_ref, a_ref, b_ref, o_ref):
        o_ref[...] = x_ref[...] + a_ref[0] + b_ref[0]

    return pl.pallas_call(
        body,
        grid=(GN,),
        in_specs=[pl.BlockSpec((R, F), lambda i: (i, 0)),
                  pl.BlockSpec((1, R, F), lambda i: (0, i, 0)),
                  pl.BlockSpec((1, R, F), lambda i: (1, i, 0))],
        out_specs=pl.BlockSpec((R, F), lambda i: (i, 0)),
        out_shape=jax.ShapeDtypeStruct((NP, F), jnp.float32),
    )(x, partials, partials)


def _k_out(x, eW1, eb1, eW2, eb2, eW3_row):
    def body(x_ref, w1_ref, b1_ref, w2_ref, b2_ref, w3_ref, o_ref):
        h = _swish(jnp.dot(x_ref[...], w1_ref[...], precision=_PREC)
                   + b1_ref[...])
        h = _swish(jnp.dot(h, w2_ref[...], precision=_PREC) + b2_ref[...])
        o_ref[...] = jnp.sum(h * w3_ref[...], axis=1, keepdims=True)

    return pl.pallas_call(
        body,
        grid=(GN,),
        in_specs=[pl.BlockSpec((R, F), lambda i: (i, 0)),
                  pl.BlockSpec((F, F), lambda i: (0, 0)),
                  pl.BlockSpec((1, F), lambda i: (0, 0)),
                  pl.BlockSpec((F, F), lambda i: (0, 0)),
                  pl.BlockSpec((1, F), lambda i: (0, 0)),
                  pl.BlockSpec((1, F), lambda i: (0, 0))],
        out_specs=pl.BlockSpec((R, 1), lambda i: (i, 0)),
        out_shape=jax.ShapeDtypeStruct((NP, 1), jnp.float32),
    )(x, eW1, eb1, eW2, eb2, eW3_row)


_NW = 32
_CH = 128


def _gather_one_call(table, idx, width=F):
    B = idx.shape[0]
    per = B // _NW
    halfA = ((per // 2) // 8) * 8
    lenB = per - halfA
    nf = min(halfA // _CH, lenB // _CH)

    def _tail_chunks(start, length):
        out = []
        done = nf * _CH
        while done < length:
            n = min(_CH, length - done)
            out.append((start + done, n))
            done += n
        return out

    mesh = plsc.VectorSubcoreMesh(core_axis_name="c", subcore_axis_name="s")

    @functools.partial(
        pl.kernel, mesh=mesh,
        out_type=jax.ShapeDtypeStruct((B, width), jnp.float32),
        scratch_types=[pltpu.VMEM((_CH,), jnp.int32),
                       pltpu.VMEM((_CH, width), jnp.float32),
                       pltpu.VMEM((_CH,), jnp.int32),
                       pltpu.VMEM((_CH, width), jnp.float32),
                       pltpu.SemaphoreType.DMA,
                       pltpu.SemaphoreType.DMA],
    )
    def k(tab, ih, oh, iva, rva, ivb, rvb, sa, sb):
        wid = lax.axis_index("s") * 2 + lax.axis_index("c")
        base = wid * per

        def do1(off, n, iv, rv, sem):
            pltpu.sync_copy(ih.at[pl.ds(off, n)], iv.at[pl.ds(0, n)])
            pltpu.async_copy(tab.at[iv.at[pl.ds(0, n)]],
                             rv.at[pl.ds(0, n)], sem).wait()
            pltpu.sync_copy(rv.at[pl.ds(0, n)], oh.at[pl.ds(off, n)])

        def do(offa, offb, n):
            pltpu.sync_copy(ih.at[pl.ds(offa, n)], iva.at[pl.ds(0, n)])
            cpa = pltpu.async_copy(tab.at[iva.at[pl.ds(0, n)]],
                                   rva.at[pl.ds(0, n)], sa)
            pltpu.sync_copy(ih.at[pl.ds(offb, n)], ivb.at[pl.ds(0, n)])
            cpb = pltpu.async_copy(tab.at[ivb.at[pl.ds(0, n)]],
                                   rvb.at[pl.ds(0, n)], sb)
            cpa.wait()
            pltpu.sync_copy(rva.at[pl.ds(0, n)], oh.at[pl.ds(offa, n)])
            cpb.wait()
            pltpu.sync_copy(rvb.at[pl.ds(0, n)], oh.at[pl.ds(offb, n)])

        def body(i, _):
            do(base + i * _CH, base + halfA + i * _CH, _CH)
            return ()

        lax.fori_loop(0, nf, body, ())
        for off, n in _tail_chunks(base, halfA):
            do1(off, n, iva, rva, sa)
        for off, n in _tail_chunks(base + halfA, lenB):
            do1(off, n, ivb, rvb, sb)

    return k(table, idx)


def _gather_pair_call(table, idx_a, idx_b):
    B = idx_a.shape[0]
    per = B // _NW
    nfull = per // _CH
    rem = per - nfull * _CH
    mesh = plsc.VectorSubcoreMesh(core_axis_name="c", subcore_axis_name="s")

    @functools.partial(
        pl.kernel, mesh=mesh,
        out_type=(jax.ShapeDtypeStruct((B, F), jnp.float32),
                  jax.ShapeDtypeStruct((B, F), jnp.float32)),
        scratch_types=[pltpu.VMEM((_CH,), jnp.int32),
                       pltpu.VMEM((_CH, F), jnp.float32),
                       pltpu.VMEM((_CH,), jnp.int32),
                       pltpu.VMEM((_CH, F), jnp.float32),
                       pltpu.SemaphoreType.DMA,
                       pltpu.SemaphoreType.DMA],
    )
    def k(tab, ia, ib, oa, ob, iva, rva, ivb, rvb, sa, sb):
        wid = lax.axis_index("s") * 2 + lax.axis_index("c")
        base = wid * per

        def do(off, n):
            pltpu.sync_copy(ia.at[pl.ds(off, n)], iva.at[pl.ds(0, n)])
            cpa = pltpu.async_copy(tab.at[iva.at[pl.ds(0, n)]],
                                   rva.at[pl.ds(0, n)], sa)
            pltpu.sync_copy(ib.at[pl.ds(off, n)], ivb.at[pl.ds(0, n)])
            cpb = pltpu.async_copy(tab.at[ivb.at[pl.ds(0, n)]],
                                   rvb.at[pl.ds(0, n)], sb)
            cpa.wait()
            pltpu.sync_copy(rva.at[pl.ds(0, n)], oa.at[pl.ds(off, n)])
            cpb.wait()
            pltpu.sync_copy(rvb.at[pl.ds(0, n)], ob.at[pl.ds(off, n)])

        def body(i, _):
            do(base + i * _CH, _CH)
            return ()

        lax.fori_loop(0, nfull, body, ())
        if rem:
            do(base + nfull * _CH, rem)

    return k(table, idx_a, idx_b)


def _scatter_add_call(msg, dst_idx, zeros_hbm):
    per_core = EP // 2
    per_sub = per_core // 16
    nf = per_sub // _CH
    rem = per_sub - nf * _CH
    rows_sub = NP // 16
    mesh = plsc.VectorSubcoreMesh(core_axis_name="c", subcore_axis_name="s")

    @functools.partial(
        pl.kernel, mesh=mesh,
        out_type=jax.ShapeDtypeStruct((2, NP, F), jnp.float32),
        scratch_types=[pltpu.VMEM((_CH,), jnp.int32),
                       pltpu.VMEM((_CH, F), jnp.float32),
                       pltpu.VMEM_SHARED((NP, F), jnp.float32)],
    )
    def k(msg_h, idx_h, zero_h, out_h, iv, rv, shared):
        c = lax.axis_index("c")
        sid = lax.axis_index("s")
        pltpu.sync_copy(zero_h.at[pl.ds(sid * rows_sub, rows_sub)],
                        shared.at[pl.ds(sid * rows_sub, rows_sub)])
        plsc.subcore_barrier()
        base = c * per_core + sid * per_sub

        def do(off, n):
            pltpu.sync_copy(idx_h.at[pl.ds(off, n)], iv.at[pl.ds(0, n)])
            pltpu.sync_copy(msg_h.at[pl.ds(off, n)], rv.at[pl.ds(0, n)])
            pltpu.sync_copy(rv.at[pl.ds(0, n)],
                            shared.at[iv.at[pl.ds(0, n)]], add=True)

        def body(i, _):
            do(base + i * _CH, _CH)
            return ()

        lax.fori_loop(0, nf, body, ())
        if rem:
            do(base + nf * _CH, rem)
        plsc.subcore_barrier()
        pltpu.sync_copy(shared.at[pl.ds(sid * rows_sub, rows_sub)],
                        out_h.at[c].at[pl.ds(sid * rows_sub, rows_sub)])

    return k(msg, dst_idx, zeros_hbm)


def _pad1(a, n, val):
    return jnp.concatenate(
        [a, jnp.full((n - a.shape[0],), val, a.dtype)])


def kernel(atomic_numbers, edge_index, edge_dist, three_body_indices, norm_ik,
           three_body_cos_angles, total_num_bonds, total_num_angles, params):
    p = params
    f32 = jnp.float32
    i32 = jnp.int32
    tbi0 = three_body_indices[:, 0].astype(i32)
    tbi1 = three_body_indices[:, 1].astype(i32)
    src = edge_index[0].astype(i32)
    dst = edge_index[1].astype(i32)

    _, norm_s, cos_s, tbi1_s = lax.sort(
        (tbi0, norm_ik.astype(f32), three_body_cos_angles.astype(f32), tbi1),
        num_keys=1)
    cnt_a = jnp.zeros((N_EDGES,), i32).at[tbi0].add(1)
    csa = jnp.cumsum(cnt_a)
    rsA_a = _pad1(jnp.concatenate([jnp.zeros((1,), i32), csa[:-1]]),
                  EP, N_ANGLES)
    rsB_a = _pad1(csa, EP, N_ANGLES)

    dist3d = _pad1(edge_dist.astype(f32), EP, 10.0).reshape(GE, 2, 128)
    norm3d = _pad1(norm_s, AP, 10.0).reshape(GA, 2, 128)
    cos3d = _pad1(cos_s, AP, 0.0).reshape(GA, 2, 128)

    eb = _k_bas_edge(dist3d)
    ab = _k_bas_ang(norm3d, cos3d)
    E0s = jnp.stack([o.reshape(EP) for o in eb], axis=1)
    E0s = jnp.concatenate([E0s, jnp.zeros((EP, 6), f32)], axis=1)
    Ps = jnp.stack([o.reshape(AP) for o in ab], axis=1)
    Ps = jnp.concatenate([Ps, jnp.zeros((AP, 6), f32)], axis=1)

    src_p = _pad1(src, EP, 0)
    dst_p = _pad1(dst, EP, 0)
    tbi1_p = _pad1(tbi1_s, AP, 0)

    emb_pad = jnp.zeros((F, F), f32).at[:NUM_EL].set(p["emb"].astype(f32))
    enc_Wp = jnp.zeros((16, F), f32).at[:N_MAX + 1].set(p["enc_W"].astype(f32))
    enc_b = p["enc_b"].astype(f32)[None, :]
    Ltri = jnp.asarray(np.tril(np.ones((2 * R, 2 * R), np.float32), -1)).astype(jnp.bfloat16)
    sa_np = np.zeros((16, 32), np.float32)
    sb_np = np.zeros((16, 32), np.float32)
    for l in range(L_MAX + 1):
        for n in range(N_MAX + 1):
            sa_np[n, l * 5 + n] = 1.0
            sb_np[5 + l, l * 5 + n] = 1.0
    SA = jnp.asarray(sa_np)
    SB = jnp.asarray(sb_np)
    zeros_np = jnp.zeros((NP, F), f32)

    blocks = p["blocks"]
    Wang_pads = [jnp.zeros((32, F), f32).at[:25].set(b["Wang"].astype(f32))
                 for b in blocks]
    WegPs = [jnp.zeros((16, F), f32).at[5:10].set(b["Weg"].astype(f32))
             for b in blocks]
    WngPs = [jnp.zeros((16, F), f32).at[5:10].set(b["Wng"].astype(f32))
             for b in blocks]

    atomic_col = _pad1(atomic_numbers.astype(i32), NP, 0)[:, None]
    x = _k_emb(atomic_col, emb_pad)
    e, t = _k_enc(E0s, enc_Wp, enc_b, blocks[0]["We3"].astype(f32))

    for b in range(NBLOCKS):
        blk = blocks[b]
        g = _gather_one_call(t, tbi1_p)
        C = _k_msg3_cumsum(Ps, g, SA, SB, Wang_pads[b], Ltri)
        Ga, Gb = _gather_pair_call(C, rsA_a, rsB_a)
        xs, xd = _gather_pair_call(x, src_p, dst_p)
        emit_t = b < NBLOCKS - 1
        We3n = (blocks[b + 1]["We3"] if emit_t else blocks[0]["We3"]).astype(f32)
        outs = _k_edge_node(Ga, Gb, e, xs, xd, E0s, blk["W3o"].astype(f32),
                            blk["Wedge"].astype(f32), blk["Wnode"].astype(f32),
                            WegPs[b], WngPs[b], We3n, emit_t)
        if emit_t:
            e, msg, t = outs
        else:
            e, msg = outs
        partials = _scatter_add_call(msg, dst_p, zeros_np)
        x = _k_xupd(x, partials)

    energy = _k_out(x, p["eW1"].astype(f32), p["eb1"].astype(f32)[None, :],
                    p["eW2"].astype(f32), p["eb2"].astype(f32)[None, :],
                    p["eW3"].astype(f32)[:, 0][None, :])
    return energy[:N_NODES] + p["eb3"].astype(f32)[None, :]


# speedup vs baseline: 3.5813x; 1.0880x over previous
import functools

import jax
import jax.numpy as jnp
import numpy as np
from jax import lax
from jax.experimental import pallas as pl
from jax.experimental.pallas import tpu as pltpu
from jax.experimental.pallas import tpu_sc as plsc

N_NODES = 10000
N_EDGES = 160000
N_ANGLES = 400000
F = 128
L_MAX = 4
N_MAX = 4
CUTOFF = 5.0
CUT3 = 4.0
NUM_EL = 108
NBLOCKS = 4

R = 256
EP = 160768
AP = 400384
NP = 10240
GE = EP // R
GA = AP // R
GN = NP // R

_PREC = jax.lax.Precision.DEFAULT


def _swish(x):
    return x * (0.5 * jnp.tanh(0.5 * x) + 0.5)


def _poly_cutoff(r, c):
    t = jnp.clip(r / c, 0.0, 1.0)
    return 1.0 - 6.0 * t ** 5 + 15.0 * t ** 4 - 10.0 * t ** 3


def _bessel_list(r, cutoff):
    r_ = r + 1e-8
    s = np.sqrt(2.0 / cutoff).astype(np.float32)
    return [s * jnp.sin((n + 1) * np.float32(np.pi) * r_ / cutoff) / r_
            for n in range(N_MAX + 1)]


def _legendre_list(c):
    polys = [jnp.ones_like(c), c]
    for l in range(2, L_MAX + 1):
        polys.append(((2 * l - 1) * c * polys[-1] - (l - 1) * polys[-2]) / l)
    return polys


def _k_bas_edge(dist3d):
    def body(r_ref, *outs):
        r = r_ref[...]
        e0 = _bessel_list(r, CUTOFF)
        fc = _poly_cutoff(r, CUTOFF)
        for n in range(N_MAX + 1):
            outs[n][...] = e0[n]
            outs[5 + n][...] = e0[n] * fc

    return pl.pallas_call(
        body,
        grid=(GE // 4,),
        in_specs=[pl.BlockSpec((4, 2, 128), lambda i: (i, 0, 0))],
        out_specs=[pl.BlockSpec((4, 2, 128), lambda i: (i, 0, 0))] * 10,
        out_shape=[jax.ShapeDtypeStruct((GE, 2, 128), jnp.float32)] * 10,
    )(dist3d)


def _k_bas_ang(norm3d, cos3d):
    def body(r_ref, c_ref, *outs):
        r = r_ref[...]
        c = c_ref[...]
        rad = _bessel_list(r, CUT3)
        leg = _legendre_list(c)
        fc3 = _poly_cutoff(r, CUT3)
        for n in range(N_MAX + 1):
            outs[n][...] = rad[n] * fc3
            outs[5 + n][...] = leg[n]

    return pl.pallas_call(
        body,
        grid=(GA // 4,),
        in_specs=[pl.BlockSpec((4, 2, 128), lambda i: (i, 0, 0))] * 2,
        out_specs=[pl.BlockSpec((4, 2, 128), lambda i: (i, 0, 0))] * 10,
        out_shape=[jax.ShapeDtypeStruct((GA, 2, 128), jnp.float32)] * 10,
    )(norm3d, cos3d)


def _k_enc(E0s, enc_Wp, enc_b, We3_0):
    def body(e0_ref, w_ref, b_ref, w3_ref, e_ref, t_ref):
        acc = jnp.dot(e0_ref[...], w_ref[...], precision=_PREC) + b_ref[...]
        e = _swish(acc)
        e_ref[...] = e
        t_ref[...] = _swish(jnp.dot(e, w3_ref[...], precision=_PREC))

    return pl.pallas_call(
        body,
        grid=(EP // 512,),
        in_specs=[pl.BlockSpec((512, 16), lambda i: (i, 0)),
                  pl.BlockSpec((16, F), lambda i: (0, 0)),
                  pl.BlockSpec((1, F), lambda i: (0, 0)),
                  pl.BlockSpec((F, F), lambda i: (0, 0))],
        out_specs=[pl.BlockSpec((512, F), lambda i: (i, 0)),
                   pl.BlockSpec((512, F), lambda i: (i, 0))],
        out_shape=[jax.ShapeDtypeStruct((EP, F), jnp.float32),
                   jax.ShapeDtypeStruct((EP, F), jnp.float32)],
    )(E0s, enc_Wp, enc_b, We3_0)


def _k_emb(atomic_col, emb_pad):
    def body(a_ref, w_ref, o_ref):
        a = a_ref[...]
        lanes = lax.broadcasted_iota(jnp.int32, (1, F), 1)
        oh = (a == lanes).astype(jnp.float32)
        o_ref[...] = jnp.dot(oh, w_ref[...], precision=_PREC)

    return pl.pallas_call(
        body,
        grid=(GN,),
        in_specs=[pl.BlockSpec((R, 1), lambda i: (i, 0)),
                  pl.BlockSpec((F, F), lambda i: (0, 0))],
        out_specs=pl.BlockSpec((R, F), lambda i: (i, 0)),
        out_shape=jax.ShapeDtypeStruct((NP, F), jnp.float32),
    )(atomic_col, emb_pad)


def _k_msg3_cumsum(P, g, SA, SB, Wang_pad, Ltri):
    def body(p_ref, g_ref, sa_ref, sb_ref, w_ref, l_ref, c_ref, carry):
        i = pl.program_id(0)

        @pl.when(i == 0)
        def _():
            carry[...] = jnp.zeros((8, F), jnp.float32)

        p = p_ref[...]
        ang = (jnp.dot(p, sa_ref[...], precision=_PREC)
               * jnp.dot(p, sb_ref[...], precision=_PREC))
        a = jnp.dot(ang, w_ref[...], precision=_PREC)
        msg = a * g_ref[...]
        cv = carry[0:1, :]
        c_ref[...] = cv + jnp.dot(l_ref[...], msg.astype(jnp.bfloat16),
                                  preferred_element_type=jnp.float32)
        carry[0:1, :] = cv + jnp.sum(msg, axis=0, keepdims=True)

    return pl.pallas_call(
        body,
        grid=(GA // 2,),
        in_specs=[pl.BlockSpec((2 * R, 16), lambda i: (i, 0)),
                  pl.BlockSpec((2 * R, F), lambda i: (i, 0)),
                  pl.BlockSpec((16, 32), lambda i: (0, 0)),
                  pl.BlockSpec((16, 32), lambda i: (0, 0)),
                  pl.BlockSpec((32, F), lambda i: (0, 0)),
                  pl.BlockSpec((2 * R, 2 * R), lambda i: (0, 0))],
        out_specs=pl.BlockSpec((2 * R, F), lambda i: (i, 0)),
        out_shape=jax.ShapeDtypeStruct((AP, F), jnp.float32),
        scratch_shapes=[pltpu.VMEM((8, F), jnp.float32)],
    )(P, g, SA, SB, Wang_pad, Ltri)


def _k_edge_node(Ga, Gb, e, xs, xd, E0s, W3o, Wedge, Wnode, WegP, WngP,
                 We3n, emit_t):
    def body(ga_ref, gb_ref, e_ref, xs_ref, xd_ref, e0_ref, w3o_ref, we_ref,
             wn_ref, weg_ref, wng_ref, we3_ref, *outs):
        if emit_t:
            e2_ref, m_ref, t_ref = outs
        else:
            e2_ref, m_ref = outs

        agg3 = gb_ref[...] - ga_ref[...]
        e1 = e_ref[...] + _swish(jnp.dot(agg3, w3o_ref[...], precision=_PREC))

        e0 = e0_ref[...]
        gate_e = jnp.dot(e0, weg_ref[...], precision=_PREC)
        gate_n = jnp.dot(e0, wng_ref[...], precision=_PREC)

        xs = xs_ref[...]
        xd = xd_ref[...]
        we = we_ref[...]
        wn = wn_ref[...]
        arg_e = (jnp.dot(xs, we[0:F, :], precision=_PREC)
                 + jnp.dot(xd, we[F:2 * F, :], precision=_PREC)
                 + jnp.dot(e1, we[2 * F:3 * F, :], precision=_PREC))
        e2 = e1 + _swish(arg_e) * gate_e
        arg_n = (jnp.dot(xs, wn[0:F, :], precision=_PREC)
                 + jnp.dot(xd, wn[F:2 * F, :], precision=_PREC)
                 + jnp.dot(e1, wn[2 * F:3 * F, :], precision=_PREC))
        msg = _swish(arg_n) * gate_n

        m_ref[...] = msg
        e2_ref[...] = e2
        if emit_t:
            t_ref[...] = _swish(jnp.dot(e2, we3_ref[...], precision=_PREC))

    n_out = 3 if emit_t else 2
    return pl.pallas_call(
        body,
        grid=(EP // 512,),
        in_specs=[pl.BlockSpec((512, F), lambda i: (i, 0)),
                  pl.BlockSpec((512, F), lambda i: (i, 0)),
                  pl.BlockSpec((512, F), lambda i: (i, 0)),
                  pl.BlockSpec((512, F), lambda i: (i, 0)),
                  pl.BlockSpec((512, F), lambda i: (i, 0)),
                  pl.BlockSpec((512, 16), lambda i: (i, 0)),
                  pl.BlockSpec((F, F), lambda i: (0, 0)),
                  pl.BlockSpec((3 * F, F), lambda i: (0, 0)),
                  pl.BlockSpec((3 * F, F), lambda i: (0, 0)),
                  pl.BlockSpec((16, F), lambda i: (0, 0)),
                  pl.BlockSpec((16, F), lambda i: (0, 0)),
                  pl.BlockSpec((F, F), lambda i: (0, 0))],
        out_specs=[pl.BlockSpec((512, F), lambda i: (i, 0))] * n_out,
        out_shape=[jax.ShapeDtypeStruct((EP, F), jnp.float32)] * n_out,
    )(Ga, Gb, e, xs, xd, E0s, W3o, Wedge, Wnode, WegP, WngP, We3n)


def _k_xupd(x, partials):
    def body(x_ref, a_ref, b_ref, o_ref):
        o_ref[...] = x_ref[...] + a_ref[0] + b_ref[0]

    return pl.pallas_call(
        body,
        grid=(GN,),
        in_specs=[pl.BlockSpec((R, F), lambda i: (i, 0)),
                  pl.BlockSpec((1, R, F), lambda i: (0, i, 0)),
                  pl.BlockSpec((1, R, F), lambda i: (1, i, 0))],
        out_specs=pl.BlockSpec((R, F), lambda i: (i, 0)),
        out_shape=jax.ShapeDtypeStruct((NP, F), jnp.float32),
    )(x, partials, partials)


def _k_out(x, eW1, eb1, eW2, eb2, eW3_row):
    def body(x_ref, w1_ref, b1_ref, w2_ref, b2_ref, w3_ref, o_ref):
        h = _swish(jnp.dot(x_ref[...], w1_ref[...], precision=_PREC)
                   + b1_ref[...])
        h = _swish(jnp.dot(h, w2_ref[...], precision=_PREC) + b2_ref[...])
        o_ref[...] = jnp.sum(h * w3_ref[...], axis=1, keepdims=True)

    return pl.pallas_call(
        body,
        grid=(GN,),
        in_specs=[pl.BlockSpec((R, F), lambda i: (i, 0)),
                  pl.BlockSpec((F, F), lambda i: (0, 0)),
                  pl.BlockSpec((1, F), lambda i: (0, 0)),
                  pl.BlockSpec((F, F), lambda i: (0, 0)),
                  pl.BlockSpec((1, F), lambda i: (0, 0)),
                  pl.BlockSpec((1, F), lambda i: (0, 0))],
        out_specs=pl.BlockSpec((R, 1), lambda i: (i, 0)),
        out_shape=jax.ShapeDtypeStruct((NP, 1), jnp.float32),
    )(x, eW1, eb1, eW2, eb2, eW3_row)


_NW = 32
_CH = 128


def _gather_one_call(table, idx, width=F):
    B = idx.shape[0]
    per = B // _NW
    halfA = ((per // 2) // 8) * 8
    lenB = per - halfA
    nf = min(halfA // _CH, lenB // _CH)

    def _tail_chunks(start, length):
        out = []
        done = nf * _CH
        while done < length:
            n = min(_CH, length - done)
            out.append((start + done, n))
            done += n
        return out

    mesh = plsc.VectorSubcoreMesh(core_axis_name="c", subcore_axis_name="s")

    @functools.partial(
        pl.kernel, mesh=mesh,
        out_type=jax.ShapeDtypeStruct((B, width), jnp.float32),
        scratch_types=[pltpu.VMEM((_CH,), jnp.int32),
                       pltpu.VMEM((_CH, width), jnp.float32),
                       pltpu.VMEM((_CH,), jnp.int32),
                       pltpu.VMEM((_CH, width), jnp.float32),
                       pltpu.SemaphoreType.DMA,
                       pltpu.SemaphoreType.DMA],
    )
    def k(tab, ih, oh, iva, rva, ivb, rvb, sa, sb):
        wid = lax.axis_index("s") * 2 + lax.axis_index("c")
        base = wid * per

        def do1(off, n, iv, rv, sem):
            pltpu.sync_copy(ih.at[pl.ds(off, n)], iv.at[pl.ds(0, n)])
            pltpu.async_copy(tab.at[iv.at[pl.ds(0, n)]],
                             rv.at[pl.ds(0, n)], sem).wait()
            pltpu.sync_copy(rv.at[pl.ds(0, n)], oh.at[pl.ds(off, n)])

        def do(offa, offb, n):
            pltpu.sync_copy(ih.at[pl.ds(offa, n)], iva.at[pl.ds(0, n)])
            cpa = pltpu.async_copy(tab.at[iva.at[pl.ds(0, n)]],
                                   rva.at[pl.ds(0, n)], sa)
            pltpu.sync_copy(ih.at[pl.ds(offb, n)], ivb.at[pl.ds(0, n)])
            cpb = pltpu.async_copy(tab.at[ivb.at[pl.ds(0, n)]],
                                   rvb.at[pl.ds(0, n)], sb)
            cpa.wait()
            pltpu.sync_copy(rva.at[pl.ds(0, n)], oh.at[pl.ds(offa, n)])
            cpb.wait()
            pltpu.sync_copy(rvb.at[pl.ds(0, n)], oh.at[pl.ds(offb, n)])

        def body(i, _):
            do(base + i * _CH, base + halfA + i * _CH, _CH)
            return ()

        lax.fori_loop(0, nf, body, ())
        for off, n in _tail_chunks(base, halfA):
            do1(off, n, iva, rva, sa)
        for off, n in _tail_chunks(base + halfA, lenB):
            do1(off, n, ivb, rvb, sb)

    return k(table, idx)


def _gather_pair_call(table, idx_a, idx_b):
    B = idx_a.shape[0]
    per = B // _NW
    nfull = per // _CH
    rem = per - nfull * _CH
    mesh = plsc.VectorSubcoreMesh(core_axis_name="c", subcore_axis_name="s")

    @functools.partial(
        pl.kernel, mesh=mesh,
        out_type=(jax.ShapeDtypeStruct((B, F), jnp.float32),
                  jax.ShapeDtypeStruct((B, F), jnp.float32)),
        scratch_types=[pltpu.VMEM((_CH,), jnp.int32),
                       pltpu.VMEM((_CH, F), jnp.float32),
                       pltpu.VMEM((_CH,), jnp.int32),
                       pltpu.VMEM((_CH, F), jnp.float32),
                       pltpu.SemaphoreType.DMA,
                       pltpu.SemaphoreType.DMA],
    )
    def k(tab, ia, ib, oa, ob, iva, rva, ivb, rvb, sa, sb):
        wid = lax.axis_index("s") * 2 + lax.axis_index("c")
        base = wid * per

        def do(off, n):
            pltpu.sync_copy(ia.at[pl.ds(off, n)], iva.at[pl.ds(0, n)])
            cpa = pltpu.async_copy(tab.at[iva.at[pl.ds(0, n)]],
                                   rva.at[pl.ds(0, n)], sa)
            pltpu.sync_copy(ib.at[pl.ds(off, n)], ivb.at[pl.ds(0, n)])
            cpb = pltpu.async_copy(tab.at[ivb.at[pl.ds(0, n)]],
                                   rvb.at[pl.ds(0, n)], sb)
            cpa.wait()
            pltpu.sync_copy(rva.at[pl.ds(0, n)], oa.at[pl.ds(off, n)])
            cpb.wait()
            pltpu.sync_copy(rvb.at[pl.ds(0, n)], ob.at[pl.ds(off, n)])

        def body(i, _):
            do(base + i * _CH, _CH)
            return ()

        lax.fori_loop(0, nfull, body, ())
        if rem:
            do(base + nfull * _CH, rem)

    return k(table, idx_a, idx_b)


def _scatter_add_call(msg, dst_idx, zeros_hbm):
    per_core = EP // 2
    per_sub = per_core // 16
    nf = per_sub // _CH
    rem = per_sub - nf * _CH
    rows_sub = NP // 16
    mesh = plsc.VectorSubcoreMesh(core_axis_name="c", subcore_axis_name="s")

    @functools.partial(
        pl.kernel, mesh=mesh,
        out_type=jax.ShapeDtypeStruct((2, NP, F), jnp.float32),
        scratch_types=[pltpu.VMEM((_CH,), jnp.int32),
                       pltpu.VMEM((_CH, F), jnp.float32),
                       pltpu.VMEM_SHARED((NP, F), jnp.float32)],
    )
    def k(msg_h, idx_h, zero_h, out_h, iv, rv, shared):
        c = lax.axis_index("c")
        sid = lax.axis_index("s")
        pltpu.sync_copy(zero_h.at[pl.ds(sid * rows_sub, rows_sub)],
                        shared.at[pl.ds(sid * rows_sub, rows_sub)])
        plsc.subcore_barrier()
        base = c * per_core + sid * per_sub

        def do(off, n):
            pltpu.sync_copy(idx_h.at[pl.ds(off, n)], iv.at[pl.ds(0, n)])
            pltpu.sync_copy(msg_h.at[pl.ds(off, n)], rv.at[pl.ds(0, n)])
            pltpu.sync_copy(rv.at[pl.ds(0, n)],
                            shared.at[iv.at[pl.ds(0, n)]], add=True)

        def body(i, _):
            do(base + i * _CH, _CH)
            return ()

        lax.fori_loop(0, nf, body, ())
        if rem:
            do(base + nf * _CH, rem)
        plsc.subcore_barrier()
        pltpu.sync_copy(shared.at[pl.ds(sid * rows_sub, rows_sub)],
                        out_h.at[c].at[pl.ds(sid * rows_sub, rows_sub)])

    return k(msg, dst_idx, zeros_hbm)


def _pad1(a, n, val):
    return jnp.concatenate(
        [a, jnp.full((n - a.shape[0],), val, a.dtype)])


def kernel(atomic_numbers, edge_index, edge_dist, three_body_indices, norm_ik,
           three_body_cos_angles, total_num_bonds, total_num_angles, params):
    p = params
    f32 = jnp.float32
    i32 = jnp.int32
    tbi0 = three_body_indices[:, 0].astype(i32)
    tbi1 = three_body_indices[:, 1].astype(i32)
    src = edge_index[0].astype(i32)
    dst = edge_index[1].astype(i32)

    _, norm_s, cos_s, tbi1_s = lax.sort(
        (tbi0, norm_ik.astype(f32), three_body_cos_angles.astype(f32), tbi1),
        num_keys=1)
    cnt_a = jnp.zeros((N_EDGES,), i32).at[tbi0].add(1)
    csa = jnp.cumsum(cnt_a)
    rsA_a = _pad1(jnp.concatenate([jnp.zeros((1,), i32), csa[:-1]]),
                  EP, N_ANGLES)
    rsB_a = _pad1(csa, EP, N_ANGLES)

    dist3d = _pad1(edge_dist.astype(f32), EP, 10.0).reshape(GE, 2, 128)
    norm3d = _pad1(norm_s, AP, 10.0).reshape(GA, 2, 128)
    cos3d = _pad1(cos_s, AP, 0.0).reshape(GA, 2, 128)

    eb = _k_bas_edge(dist3d)
    ab = _k_bas_ang(norm3d, cos3d)
    E0s = jnp.stack([o.reshape(EP) for o in eb], axis=1)
    E0s = jnp.concatenate([E0s, jnp.zeros((EP, 6), f32)], axis=1)
    Ps = jnp.stack([o.reshape(AP) for o in ab], axis=1)
    Ps = jnp.concatenate([Ps, jnp.zeros((AP, 6), f32)], axis=1)

    src_p = _pad1(src, EP, 0)
    dst_p = _pad1(dst, EP, 0)
    tbi1_p = _pad1(tbi1_s, AP, 0)

    emb_pad = jnp.zeros((F, F), f32).at[:NUM_EL].set(p["emb"].astype(f32))
    enc_Wp = jnp.zeros((16, F), f32).at[:N_MAX + 1].set(p["enc_W"].astype(f32))
    enc_b = p["enc_b"].astype(f32)[None, :]
    Ltri = jnp.asarray(np.tril(np.ones((2 * R, 2 * R), np.float32), -1)).astype(jnp.bfloat16)
    sa_np = np.zeros((16, 32), np.float32)
    sb_np = np.zeros((16, 32), np.float32)
    for l in range(L_MAX + 1):
        for n in range(N_MAX + 1):
            sa_np[n, l * 5 + n] = 1.0
            sb_np[5 + l, l * 5 + n] = 1.0
    SA = jnp.asarray(sa_np)
    SB = jnp.asarray(sb_np)
    zeros_np = jnp.zeros((NP, F), f32)

    blocks = p["blocks"]
    Wang_pads = [jnp.zeros((32, F), f32).at[:25].set(b["Wang"].astype(f32))
                 for b in blocks]
    WegPs = [jnp.zeros((16, F), f32).at[5:10].set(b["Weg"].astype(f32))
             for b in blocks]
    WngPs = [jnp.zeros((16, F), f32).at[5:10].set(b["Wng"].astype(f32))
             for b in blocks]

    atomic_col = _pad1(atomic_numbers.astype(i32), NP, 0)[:, None]
    x = _k_emb(atomic_col, emb_pad)
    e, t = _k_enc(E0s, enc_Wp, enc_b, blocks[0]["We3"].astype(f32))

    for b in range(NBLOCKS):
        blk = blocks[b]
        g = _gather_one_call(t, tbi1_p)
        C = _k_msg3_cumsum(Ps, g, SA, SB, Wang_pads[b], Ltri)
        Ga, Gb = _gather_pair_call(C, rsA_a, rsB_a)
        xs, xd = _gather_pair_call(x, src_p, dst_p)
        emit_t = b < NBLOCKS - 1
        We3n = (blocks[b + 1]["We3"] if emit_t else blocks[0]["We3"]).astype(f32)
        outs = _k_edge_node(Ga, Gb, e, xs, xd, E0s, blk["W3o"].astype(f32),
                            blk["Wedge"].astype(f32), blk["Wnode"].astype(f32),
                            WegPs[b], WngPs[b], We3n, emit_t)
        if emit_t:
            e, msg, t = outs
        else:
            e, msg = outs
        partials = _scatter_add_call(msg, dst_p, zeros_np)
        x = _k_xupd(x, partials)

    energy = _k_out(x, p["eW1"].astype(f32), p["eb1"].astype(f32)[None, :],
                    p["eW2"].astype(f32), p["eb2"].astype(f32)[None, :],
                    p["eW3"].astype(f32)[:, 0][None, :])
    return energy[:N_NODES] + p["eb3"].astype(f32)[None, :]


# speedup vs baseline: 3.5863x; 1.0014x over previous
import functools

import jax
import jax.numpy as jnp
import numpy as np
from jax import lax
from jax.experimental import pallas as pl
from jax.experimental.pallas import tpu as pltpu
from jax.experimental.pallas import tpu_sc as plsc

N_NODES = 10000
N_EDGES = 160000
N_ANGLES = 400000
F = 128
L_MAX = 4
N_MAX = 4
CUTOFF = 5.0
CUT3 = 4.0
NUM_EL = 108
NBLOCKS = 4

R = 256
EP = 160768
AP = 400384
NP = 10240
GE = EP // R
GA = AP // R
GN = NP // R

_PREC = jax.lax.Precision.DEFAULT


def _swish(x):
    return x * (0.5 * jnp.tanh(0.5 * x) + 0.5)


def _poly_cutoff(r, c):
    t = jnp.clip(r / c, 0.0, 1.0)
    return 1.0 - 6.0 * t ** 5 + 15.0 * t ** 4 - 10.0 * t ** 3


def _bessel_list(r, cutoff):
    r_ = r + 1e-8
    s = np.sqrt(2.0 / cutoff).astype(np.float32)
    return [s * jnp.sin((n + 1) * np.float32(np.pi) * r_ / cutoff) / r_
            for n in range(N_MAX + 1)]


def _legendre_list(c):
    polys = [jnp.ones_like(c), c]
    for l in range(2, L_MAX + 1):
        polys.append(((2 * l - 1) * c * polys[-1] - (l - 1) * polys[-2]) / l)
    return polys


def _k_bas_edge(dist3d):
    def body(r_ref, *outs):
        r = r_ref[...]
        e0 = _bessel_list(r, CUTOFF)
        fc = _poly_cutoff(r, CUTOFF)
        for n in range(N_MAX + 1):
            outs[n][...] = e0[n]
            outs[5 + n][...] = e0[n] * fc

    return pl.pallas_call(
        body,
        grid=(GE // 4,),
        in_specs=[pl.BlockSpec((4, 2, 128), lambda i: (i, 0, 0))],
        out_specs=[pl.BlockSpec((4, 2, 128), lambda i: (i, 0, 0))] * 10,
        out_shape=[jax.ShapeDtypeStruct((GE, 2, 128), jnp.float32)] * 10,
    )(dist3d)


def _k_bas_ang(norm3d, cos3d):
    def body(r_ref, c_ref, *outs):
        r = r_ref[...]
        c = c_ref[...]
        rad = _bessel_list(r, CUT3)
        leg = _legendre_list(c)
        fc3 = _poly_cutoff(r, CUT3)
        for n in range(N_MAX + 1):
            outs[n][...] = rad[n] * fc3
            outs[5 + n][...] = leg[n]

    return pl.pallas_call(
        body,
        grid=(GA // 4,),
        in_specs=[pl.BlockSpec((4, 2, 128), lambda i: (i, 0, 0))] * 2,
        out_specs=[pl.BlockSpec((4, 2, 128), lambda i: (i, 0, 0))] * 10,
        out_shape=[jax.ShapeDtypeStruct((GA, 2, 128), jnp.float32)] * 10,
    )(norm3d, cos3d)


def _k_enc(E0s, enc_Wp, enc_b, We3_0):
    def body(e0_ref, w_ref, b_ref, w3_ref, e_ref, t_ref):
        acc = jnp.dot(e0_ref[...], w_ref[...], precision=_PREC) + b_ref[...]
        e = _swish(acc)
        e_ref[...] = e
        t_ref[...] = _swish(jnp.dot(e, w3_ref[...], precision=_PREC))

    return pl.pallas_call(
        body,
        grid=(EP // 512,),
        in_specs=[pl.BlockSpec((512, 16), lambda i: (i, 0)),
                  pl.BlockSpec((16, F), lambda i: (0, 0)),
                  pl.BlockSpec((1, F), lambda i: (0, 0)),
                  pl.BlockSpec((F, F), lambda i: (0, 0))],
        out_specs=[pl.BlockSpec((512, F), lambda i: (i, 0)),
                   pl.BlockSpec((512, F), lambda i: (i, 0))],
        out_shape=[jax.ShapeDtypeStruct((EP, F), jnp.float32),
                   jax.ShapeDtypeStruct((EP, F), jnp.float32)],
    )(E0s, enc_Wp, enc_b, We3_0)


def _k_emb(atomic_col, emb_pad):
    def body(a_ref, w_ref, o_ref):
        a = a_ref[...]
        lanes = lax.broadcasted_iota(jnp.int32, (1, F), 1)
        oh = (a == lanes).astype(jnp.float32)
        o_ref[...] = jnp.dot(oh, w_ref[...], precision=_PREC)

    return pl.pallas_call(
        body,
        grid=(GN,),
        in_specs=[pl.BlockSpec((R, 1), lambda i: (i, 0)),
                  pl.BlockSpec((F, F), lambda i: (0, 0))],
        out_specs=pl.BlockSpec((R, F), lambda i: (i, 0)),
        out_shape=jax.ShapeDtypeStruct((NP, F), jnp.float32),
    )(atomic_col, emb_pad)


def _k_msg3_cumsum(P, g, SA, SB, Wang_pad, Ltri):
    def body(p_ref, g_ref, sa_ref, sb_ref, w_ref, l_ref, c_ref, carry):
        i = pl.program_id(0)

        @pl.when(i == 0)
        def _():
            carry[...] = jnp.zeros((8, F), jnp.float32)

        p = p_ref[...]
        ang = (jnp.dot(p, sa_ref[...], precision=_PREC)
               * jnp.dot(p, sb_ref[...], precision=_PREC))
        a = jnp.dot(ang, w_ref[...], precision=_PREC)
        msg = a * g_ref[...]
        cv = carry[0:1, :]
        c_ref[...] = cv + jnp.dot(l_ref[...], msg.astype(jnp.bfloat16),
                                  preferred_element_type=jnp.float32)
        carry[0:1, :] = cv + jnp.sum(msg, axis=0, keepdims=True)

    return pl.pallas_call(
        body,
        grid=(GA // 2,),
        in_specs=[pl.BlockSpec((2 * R, 16), lambda i: (i, 0)),
                  pl.BlockSpec((2 * R, F), lambda i: (i, 0)),
                  pl.BlockSpec((16, 32), lambda i: (0, 0)),
                  pl.BlockSpec((16, 32), lambda i: (0, 0)),
                  pl.BlockSpec((32, F), lambda i: (0, 0)),
                  pl.BlockSpec((2 * R, 2 * R), lambda i: (0, 0))],
        out_specs=pl.BlockSpec((2 * R, F), lambda i: (i, 0)),
        out_shape=jax.ShapeDtypeStruct((AP, F), jnp.float32),
        scratch_shapes=[pltpu.VMEM((8, F), jnp.float32)],
    )(P, g, SA, SB, Wang_pad, Ltri)


def _k_edge_node(Ga, Gb, e, xs, xd, E0s, W3o, Wedge, Wnode, WegP, WngP,
                 We3n, emit_t):
    def body(ga_ref, gb_ref, e_ref, xs_ref, xd_ref, e0_ref, w3o_ref, we_ref,
             wn_ref, weg_ref, wng_ref, we3_ref, *outs):
        if emit_t:
            e2_ref, m_ref, t_ref = outs
        else:
            e2_ref, m_ref = outs

        agg3 = gb_ref[...] - ga_ref[...]
        e1 = e_ref[...] + _swish(jnp.dot(agg3, w3o_ref[...], precision=_PREC))

        e0 = e0_ref[...]
        gate_e = jnp.dot(e0, weg_ref[...], precision=_PREC)
        gate_n = jnp.dot(e0, wng_ref[...], precision=_PREC)

        xs = xs_ref[...]
        xd = xd_ref[...]
        we = we_ref[...]
        wn = wn_ref[...]
        arg_e = (jnp.dot(xs, we[0:F, :], precision=_PREC)
                 + jnp.dot(xd, we[F:2 * F, :], precision=_PREC)
                 + jnp.dot(e1, we[2 * F:3 * F, :], precision=_PREC))
        e2 = e1 + _swish(arg_e) * gate_e
        arg_n = (jnp.dot(xs, wn[0:F, :], precision=_PREC)
                 + jnp.dot(xd, wn[F:2 * F, :], precision=_PREC)
                 + jnp.dot(e1, wn[2 * F:3 * F, :], precision=_PREC))
        msg = _swish(arg_n) * gate_n

        m_ref[...] = msg
        e2_ref[...] = e2
        if emit_t:
            t_ref[...] = _swish(jnp.dot(e2, we3_ref[...], precision=_PREC))

    n_out = 3 if emit_t else 2
    return pl.pallas_call(
        body,
        grid=(EP // 512,),
        in_specs=[pl.BlockSpec((512, F), lambda i: (i, 0)),
                  pl.BlockSpec((512, F), lambda i: (i, 0)),
                  pl.BlockSpec((512, F), lambda i: (i, 0)),
                  pl.BlockSpec((512, F), lambda i: (i, 0)),
                  pl.BlockSpec((512, F), lambda i: (i, 0)),
                  pl.BlockSpec((512, 16), lambda i: (i, 0)),
                  pl.BlockSpec((F, F), lambda i: (0, 0)),
                  pl.BlockSpec((3 * F, F), lambda i: (0, 0)),
                  pl.BlockSpec((3 * F, F), lambda i: (0, 0)),
                  pl.BlockSpec((16, F), lambda i: (0, 0)),
                  pl.BlockSpec((16, F), lambda i: (0, 0)),
                  pl.BlockSpec((F, F), lambda i: (0, 0))],
        out_specs=[pl.BlockSpec((512, F), lambda i: (i, 0))] * n_out,
        out_shape=[jax.ShapeDtypeStruct((EP, F), jnp.float32)] * n_out,
    )(Ga, Gb, e, xs, xd, E0s, W3o, Wedge, Wnode, WegP, WngP, We3n)


def _k_xupd(x, partials):
    def body(x_ref, a_ref, b_ref, o_ref):
        o_ref[...] = x_ref[...] + a_ref[0] + b_ref[0]

    return pl.pallas_call(
        body,
        grid=(GN,),
        in_specs=[pl.BlockSpec((R, F), lambda i: (i, 0)),
                  pl.BlockSpec((1, R, F), lambda i: (0, i, 0)),
                  pl.BlockSpec((1, R, F), lambda i: (1, i, 0))],
        out_specs=pl.BlockSpec((R, F), lambda i: (i, 0)),
        out_shape=jax.ShapeDtypeStruct((NP, F), jnp.float32),
    )(x, partials, partials)


def _k_out(x, eW1, eb1, eW2, eb2, eW3_row):
    def body(x_ref, w1_ref, b1_ref, w2_ref, b2_ref, w3_ref, o_ref):
        h = _swish(jnp.dot(x_ref[...], w1_ref[...], precision=_PREC)
                   + b1_ref[...])
        h = _swish(jnp.dot(h, w2_ref[...], precision=_PREC) + b2_ref[...])
        o_ref[...] = jnp.sum(h * w3_ref[...], axis=1, keepdims=True)

    return pl.pallas_call(
        body,
        grid=(GN,),
        in_specs=[pl.BlockSpec((R, F), lambda i: (i, 0)),
                  pl.BlockSpec((F, F), lambda i: (0, 0)),
                  pl.BlockSpec((1, F), lambda i: (0, 0)),
                  pl.BlockSpec((F, F), lambda i: (0, 0)),
                  pl.BlockSpec((1, F), lambda i: (0, 0)),
                  pl.BlockSpec((1, F), lambda i: (0, 0))],
        out_specs=pl.BlockSpec((R, 1), lambda i: (i, 0)),
        out_shape=jax.ShapeDtypeStruct((NP, 1), jnp.float32),
    )(x, eW1, eb1, eW2, eb2, eW3_row)


_NW = 32
_CH = 128


def _gather_one_call(table, idx, width=F):
    B = idx.shape[0]
    per = B // _NW
    halfA = ((per // 2) // 8) * 8
    lenB = per - halfA
    nf = min(halfA // _CH, lenB // _CH)

    def _tail_chunks(start, length):
        out = []
        done = nf * _CH
        while done < length:
            n = min(_CH, length - done)
            out.append((start + done, n))
            done += n
        return out

    mesh = plsc.VectorSubcoreMesh(core_axis_name="c", subcore_axis_name="s")

    @functools.partial(
        pl.kernel, mesh=mesh,
        out_type=jax.ShapeDtypeStruct((B, width), jnp.float32),
        scratch_types=[pltpu.VMEM((_CH,), jnp.int32),
                       pltpu.VMEM((_CH, width), jnp.float32),
                       pltpu.VMEM((_CH,), jnp.int32),
                       pltpu.VMEM((_CH, width), jnp.float32),
                       pltpu.SemaphoreType.DMA,
                       pltpu.SemaphoreType.DMA,
                       pltpu.SemaphoreType.DMA,
                       pltpu.SemaphoreType.DMA],
    )
    def k(tab, ih, oh, iva, rva, ivb, rvb, sa, sb, wa, wb):
        wid = lax.axis_index("s") * 2 + lax.axis_index("c")
        base = wid * per

        def do1(off, n, iv, rv, sem):
            pltpu.sync_copy(ih.at[pl.ds(off, n)], iv.at[pl.ds(0, n)])
            pltpu.async_copy(tab.at[iv.at[pl.ds(0, n)]],
                             rv.at[pl.ds(0, n)], sem).wait()
            pltpu.sync_copy(rv.at[pl.ds(0, n)], oh.at[pl.ds(off, n)])

        def body(i, _):
            offa = base + i * _CH
            offb = base + halfA + i * _CH

            @pl.when(i > 0)
            def _():
                pltpu.make_async_copy(
                    rva, oh.at[pl.ds(offa - _CH, _CH)], wa).wait()
                pltpu.make_async_copy(
                    rvb, oh.at[pl.ds(offb - _CH, _CH)], wb).wait()

            pltpu.sync_copy(ih.at[pl.ds(offa, _CH)], iva)
            cpa = pltpu.async_copy(tab.at[iva], rva, sa)
            pltpu.sync_copy(ih.at[pl.ds(offb, _CH)], ivb)
            cpb = pltpu.async_copy(tab.at[ivb], rvb, sb)
            cpa.wait()
            pltpu.async_copy(rva, oh.at[pl.ds(offa, _CH)], wa)
            cpb.wait()
            pltpu.async_copy(rvb, oh.at[pl.ds(offb, _CH)], wb)
            return ()

        lax.fori_loop(0, nf, body, ())
        if nf > 0:
            pltpu.make_async_copy(
                rva, oh.at[pl.ds(base + (nf - 1) * _CH, _CH)], wa).wait()
            pltpu.make_async_copy(
                rvb, oh.at[pl.ds(base + halfA + (nf - 1) * _CH, _CH)],
                wb).wait()
        for off, n in _tail_chunks(base, halfA):
            do1(off, n, iva, rva, sa)
        for off, n in _tail_chunks(base + halfA, lenB):
            do1(off, n, ivb, rvb, sb)

    return k(table, idx)


def _gather_pair_call(table, idx_a, idx_b):
    B = idx_a.shape[0]
    per = B // _NW
    nfull = per // _CH
    rem = per - nfull * _CH
    mesh = plsc.VectorSubcoreMesh(core_axis_name="c", subcore_axis_name="s")

    @functools.partial(
        pl.kernel, mesh=mesh,
        out_type=(jax.ShapeDtypeStruct((B, F), jnp.float32),
                  jax.ShapeDtypeStruct((B, F), jnp.float32)),
        scratch_types=[pltpu.VMEM((_CH,), jnp.int32),
                       pltpu.VMEM((_CH, F), jnp.float32),
                       pltpu.VMEM((_CH,), jnp.int32),
                       pltpu.VMEM((_CH, F), jnp.float32),
                       pltpu.SemaphoreType.DMA,
                       pltpu.SemaphoreType.DMA,
                       pltpu.SemaphoreType.DMA,
                       pltpu.SemaphoreType.DMA],
    )
    def k(tab, ia, ib, oa, ob, iva, rva, ivb, rvb, sa, sb, wa, wb):
        wid = lax.axis_index("s") * 2 + lax.axis_index("c")
        base = wid * per

        def body(i, _):
            off = base + i * _CH

            @pl.when(i > 0)
            def _():
                pltpu.make_async_copy(
                    rva, oa.at[pl.ds(off - _CH, _CH)], wa).wait()
                pltpu.make_async_copy(
                    rvb, ob.at[pl.ds(off - _CH, _CH)], wb).wait()

            pltpu.sync_copy(ia.at[pl.ds(off, _CH)], iva)
            cpa = pltpu.async_copy(tab.at[iva], rva, sa)
            pltpu.sync_copy(ib.at[pl.ds(off, _CH)], ivb)
            cpb = pltpu.async_copy(tab.at[ivb], rvb, sb)
            cpa.wait()
            pltpu.async_copy(rva, oa.at[pl.ds(off, _CH)], wa)
            cpb.wait()
            pltpu.async_copy(rvb, ob.at[pl.ds(off, _CH)], wb)
            return ()

        lax.fori_loop(0, nfull, body, ())
        if nfull > 0:
            last = base + (nfull - 1) * _CH
            pltpu.make_async_copy(rva, oa.at[pl.ds(last, _CH)], wa).wait()
            pltpu.make_async_copy(rvb, ob.at[pl.ds(last, _CH)], wb).wait()
        if rem:
            off = base + nfull * _CH
            pltpu.sync_copy(ia.at[pl.ds(off, rem)], iva.at[pl.ds(0, rem)])
            cpa = pltpu.async_copy(tab.at[iva.at[pl.ds(0, rem)]],
                                   rva.at[pl.ds(0, rem)], sa)
            pltpu.sync_copy(ib.at[pl.ds(off, rem)], ivb.at[pl.ds(0, rem)])
            cpb = pltpu.async_copy(tab.at[ivb.at[pl.ds(0, rem)]],
                                   rvb.at[pl.ds(0, rem)], sb)
            cpa.wait()
            pltpu.sync_copy(rva.at[pl.ds(0, rem)], oa.at[pl.ds(off, rem)])
            cpb.wait()
            pltpu.sync_copy(rvb.at[pl.ds(0, rem)], ob.at[pl.ds(off, rem)])

    return k(table, idx_a, idx_b)


def _scatter_add_call(msg, dst_idx, zeros_hbm):
    per_core = EP // 2
    per_sub = per_core // 16
    nf = per_sub // _CH
    rem = per_sub - nf * _CH
    rows_sub = NP // 16
    mesh = plsc.VectorSubcoreMesh(core_axis_name="c", subcore_axis_name="s")

    @functools.partial(
        pl.kernel, mesh=mesh,
        out_type=jax.ShapeDtypeStruct((2, NP, F), jnp.float32),
        scratch_types=[pltpu.VMEM((_CH,), jnp.int32),
                       pltpu.VMEM((_CH, F), jnp.float32),
                       pltpu.VMEM_SHARED((NP, F), jnp.float32)],
    )
    def k(msg_h, idx_h, zero_h, out_h, iv, rv, shared):
        c = lax.axis_index("c")
        sid = lax.axis_index("s")
        pltpu.sync_copy(zero_h.at[pl.ds(sid * rows_sub, rows_sub)],
                        shared.at[pl.ds(sid * rows_sub, rows_sub)])
        plsc.subcore_barrier()
        base = c * per_core + sid * per_sub

        def do(off, n):
            pltpu.sync_copy(idx_h.at[pl.ds(off, n)], iv.at[pl.ds(0, n)])
            pltpu.sync_copy(msg_h.at[pl.ds(off, n)], rv.at[pl.ds(0, n)])
            pltpu.sync_copy(rv.at[pl.ds(0, n)],
                            shared.at[iv.at[pl.ds(0, n)]], add=True)

        def body(i, _):
            do(base + i * _CH, _CH)
            return ()

        lax.fori_loop(0, nf, body, ())
        if rem:
            do(base + nf * _CH, rem)
        plsc.subcore_barrier()
        pltpu.sync_copy(shared.at[pl.ds(sid * rows_sub, rows_sub)],
                        out_h.at[c].at[pl.ds(sid * rows_sub, rows_sub)])

    return k(msg, dst_idx, zeros_hbm)


def _pad1(a, n, val):
    return jnp.concatenate(
        [a, jnp.full((n - a.shape[0],), val, a.dtype)])


def kernel(atomic_numbers, edge_index, edge_dist, three_body_indices, norm_ik,
           three_body_cos_angles, total_num_bonds, total_num_angles, params):
    p = params
    f32 = jnp.float32
    i32 = jnp.int32
    tbi0 = three_body_indices[:, 0].astype(i32)
    tbi1 = three_body_indices[:, 1].astype(i32)
    src = edge_index[0].astype(i32)
    dst = edge_index[1].astype(i32)

    _, norm_s, cos_s, tbi1_s = lax.sort(
        (tbi0, norm_ik.astype(f32), three_body_cos_angles.astype(f32), tbi1),
        num_keys=1)
    cnt_a = jnp.zeros((N_EDGES,), i32).at[tbi0].add(1)
    csa = jnp.cumsum(cnt_a)
    rsA_a = _pad1(jnp.concatenate([jnp.zeros((1,), i32), csa[:-1]]),
                  EP, N_ANGLES)
    rsB_a = _pad1(csa, EP, N_ANGLES)

    dist3d = _pad1(edge_dist.astype(f32), EP, 10.0).reshape(GE, 2, 128)
    norm3d = _pad1(norm_s, AP, 10.0).reshape(GA, 2, 128)
    cos3d = _pad1(cos_s, AP, 0.0).reshape(GA, 2, 128)

    eb = _k_bas_edge(dist3d)
    ab = _k_bas_ang(norm3d, cos3d)
    E0s = jnp.stack([o.reshape(EP) for o in eb], axis=1)
    E0s = jnp.concatenate([E0s, jnp.zeros((EP, 6), f32)], axis=1)
    Ps = jnp.stack([o.reshape(AP) for o in ab], axis=1)
    Ps = jnp.concatenate([Ps, jnp.zeros((AP, 6), f32)], axis=1)

    src_p = _pad1(src, EP, 0)
    dst_p = _pad1(dst, EP, 0)
    tbi1_p = _pad1(tbi1_s, AP, 0)

    emb_pad = jnp.zeros((F, F), f32).at[:NUM_EL].set(p["emb"].astype(f32))
    enc_Wp = jnp.zeros((16, F), f32).at[:N_MAX + 1].set(p["enc_W"].astype(f32))
    enc_b = p["enc_b"].astype(f32)[None, :]
    Ltri = jnp.asarray(np.tril(np.ones((2 * R, 2 * R), np.float32), -1)).astype(jnp.bfloat16)
    sa_np = np.zeros((16, 32), np.float32)
    sb_np = np.zeros((16, 32), np.float32)
    for l in range(L_MAX + 1):
        for n in range(N_MAX + 1):
            sa_np[n, l * 5 + n] = 1.0
            sb_np[5 + l, l * 5 + n] = 1.0
    SA = jnp.asarray(sa_np)
    SB = jnp.asarray(sb_np)
    zeros_np = jnp.zeros((NP, F), f32)

    blocks = p["blocks"]
    Wang_pads = [jnp.zeros((32, F), f32).at[:25].set(b["Wang"].astype(f32))
                 for b in blocks]
    WegPs = [jnp.zeros((16, F), f32).at[5:10].set(b["Weg"].astype(f32))
             for b in blocks]
    WngPs = [jnp.zeros((16, F), f32).at[5:10].set(b["Wng"].astype(f32))
             for b in blocks]

    atomic_col = _pad1(atomic_numbers.astype(i32), NP, 0)[:, None]
    x = _k_emb(atomic_col, emb_pad)
    e, t = _k_enc(E0s, enc_Wp, enc_b, blocks[0]["We3"].astype(f32))

    for b in range(NBLOCKS):
        blk = blocks[b]
        g = _gather_one_call(t, tbi1_p)
        C = _k_msg3_cumsum(Ps, g, SA, SB, Wang_pads[b], Ltri)
        Ga, Gb = _gather_pair_call(C, rsA_a, rsB_a)
        xs, xd = _gather_pair_call(x, src_p, dst_p)
        emit_t = b < NBLOCKS - 1
        We3n = (blocks[b + 1]["We3"] if emit_t else blocks[0]["We3"]).astype(f32)
        outs = _k_edge_node(Ga, Gb, e, xs, xd, E0s, blk["W3o"].astype(f32),
                            blk["Wedge"].astype(f32), blk["Wnode"].astype(f32),
                            WegPs[b], WngPs[b], We3n, emit_t)
        if emit_t:
            e, msg, t = outs
        else:
            e, msg = outs
        partials = _scatter_add_call(msg, dst_p, zeros_np)
        x = _k_xupd(x, partials)

    energy = _k_out(x, p["eW1"].astype(f32), p["eb1"].astype(f32)[None, :],
                    p["eW2"].astype(f32), p["eb2"].astype(f32)[None, :],
                    p["eW3"].astype(f32)[:, 0][None, :])
    return energy[:N_NODES] + p["eb3"].astype(f32)[None, :]
